# Initial kernel scaffold; baseline (speedup 1.0000x reference)
import functools

import jax
import jax.numpy as jnp
from jax import lax
from jax.experimental import pallas as pl
from jax.experimental.pallas import tpu as pltpu
from jax.experimental.pallas import tpu_sc as plsc

_VOXEL = 0.1
_SIDE = 11
_NBATCH = 16
_M = _NBATCH * _SIDE ** 3
_MP = 21504
_BM = _MP // 16
_R = 1024
_NW = 32

_mesh = plsc.VectorSubcoreMesh(core_axis_name="c", subcore_axis_name="s")


def _k1_body(nreal, pos_ref, batch_ref, clu_ref, p4_ref):
    i = pl.program_id(0)
    pos = pos_ref[...]
    coords = jnp.round(pos / _VOXEL).astype(jnp.int32)
    b = batch_ref[...]
    key = ((b * _SIDE + coords[:, 0:1]) * _SIDE
           + coords[:, 1:2]) * _SIDE + coords[:, 2:3]
    row = i * _R + lax.broadcasted_iota(jnp.int32, (_R, 1), 0)
    valid = row < nreal
    clu_ref[...] = jnp.where(valid, key, 0)
    p4 = jnp.concatenate([pos, jnp.ones((_R, 1), jnp.float32)], axis=1)
    p4_ref[...] = jnp.where(valid, p4, 0.0)


def _k1(pos, batch2, npad):
    grid = npad // _R
    return pl.pallas_call(
        functools.partial(_k1_body, pos.shape[0]),
        grid=(grid,),
        in_specs=[pl.BlockSpec((_R, 3), lambda i: (i, 0)),
                  pl.BlockSpec((_R, 1), lambda i: (i, 0))],
        out_specs=[pl.BlockSpec((_R, 1), lambda i: (i, 0)),
                   pl.BlockSpec((_R, 4), lambda i: (i, 0))],
        out_shape=[jax.ShapeDtypeStruct((npad, 1), jnp.int32),
                   jax.ShapeDtypeStruct((npad, 4), jnp.float32)],
    )(pos, batch2)


def _k2(p4, clu, zeros4):
    npad = clu.shape[0]
    pw = npad // _NW

    @functools.partial(
        pl.kernel,
        out_type=jax.ShapeDtypeStruct((2, _MP, 4), jnp.float32),
        mesh=_mesh,
        scratch_types=[pltpu.VMEM((pw,), jnp.int32),
                       pltpu.VMEM((pw, 4), jnp.float32),
                       pltpu.VMEM_SHARED((_MP, 4), jnp.float32)],
    )
    def k(p4_hbm, clu_hbm, z_hbm, out_hbm, idx_v, val_v, acc_s):
        c = lax.axis_index("c")
        s = lax.axis_index("s")

        @pl.when(s == 0)
        def _():
            pltpu.sync_copy(z_hbm, acc_s)

        plsc.subcore_barrier()
        base = (s * 2 + c) * pw
        pltpu.sync_copy(clu_hbm.at[pl.ds(base, pw)], idx_v)
        pltpu.sync_copy(p4_hbm.at[pl.ds(base, pw)], val_v)
        pltpu.sync_copy(val_v, acc_s.at[idx_v], add=True)
        plsc.subcore_barrier()

        @pl.when(s == 0)
        def _():
            pltpu.sync_copy(acc_s, out_hbm.at[c])

    return k(p4, clu, zeros4)


def _k3_body(parts_ref, wc_ref, g1_ref, aux_ref):
    p = parts_ref[...]
    ssum = p[0] + p[1]
    cnt = jnp.maximum(ssum[:, 3:4], 1.0)
    inv = 1.0 / cnt
    pp = ssum[:, 0:3] * inv
    e = wc_ref[...]
    g1_ref[...] = (pp[:, 0:1] * e[0:1] + pp[:, 1:2] * e[1:2]
                   + pp[:, 2:3] * e[2:3])
    aux_ref[...] = jnp.concatenate([pp, inv], axis=1)


def _k3(parts4, wc):
    return pl.pallas_call(
        _k3_body,
        grid=(16,),
        in_specs=[pl.BlockSpec((2, _BM, 4), lambda i: (0, i, 0)),
                  pl.BlockSpec((8, 128), lambda i: (0, 0))],
        out_specs=[pl.BlockSpec((_BM, 128), lambda i: (i, 0)),
                   pl.BlockSpec((_BM, 4), lambda i: (i, 0))],
        out_shape=[jax.ShapeDtypeStruct((_MP, 128), jnp.float32),
                   jax.ShapeDtypeStruct((_MP, 4), jnp.float32)],
    )(parts4, wc)


def _sc_gather(tab, clu):
    npad = clu.shape[0]
    pw = npad // _NW
    ch = pw // 4

    @functools.partial(
        pl.kernel,
        out_type=jax.ShapeDtypeStruct((npad, 128), jnp.float32),
        mesh=_mesh,
        scratch_types=[pltpu.VMEM((ch,), jnp.int32),
                       pltpu.VMEM((ch, 128), jnp.float32),
                       pltpu.SemaphoreType.DMA],
    )
    def k(tab_hbm, clu_hbm, out_hbm, idx_v, rows_v, sem):
        c = lax.axis_index("c")
        s = lax.axis_index("s")
        base = (s * 2 + c) * pw
        for j in range(4):
            off = base + j * ch
            pltpu.sync_copy(clu_hbm.at[pl.ds(off, ch)], idx_v)
            pltpu.async_copy(tab_hbm.at[idx_v], rows_v, sem).wait()
            pltpu.sync_copy(rows_v, out_hbm.at[pl.ds(off, ch)])

    return k(tab, clu)


def _k5_body(nreal, x_ref, p4_ref, g_ref, wa_ref, w2_ref, e_ref, out_ref):
    i = pl.program_id(0)
    e = e_ref[...]
    p = p4_ref[...]
    t = jnp.dot(x_ref[...], wa_ref[...], preferred_element_type=jnp.float32)
    t = t + g_ref[...] + e[3:4]
    t = t + p[:, 0:1] * e[0:1] + p[:, 1:2] * e[1:2] + p[:, 2:3] * e[2:3]
    a = jnp.maximum(t, 0.0)
    h = jnp.dot(a, w2_ref[...], preferred_element_type=jnp.float32) + e[4:5]
    h = jnp.maximum(h, 0.0)
    row = i * _R + lax.broadcasted_iota(jnp.int32, (_R, 1), 0)
    h = jnp.where(row < nreal, h, 0.0)
    out_ref[0] = h[:, :64]
    out_ref[1] = h[:, 64:]


def _k5(x, p4, g1c, wa, w2, e, npad):
    grid = npad // _R
    return pl.pallas_call(
        functools.partial(_k5_body, x.shape[0]),
        grid=(grid,),
        in_specs=[pl.BlockSpec((_R, 128), lambda i: (i, 0)),
                  pl.BlockSpec((_R, 4), lambda i: (i, 0)),
                  pl.BlockSpec((_R, 128), lambda i: (i, 0)),
                  pl.BlockSpec((128, 128), lambda i: (0, 0)),
                  pl.BlockSpec((128, 128), lambda i: (0, 0)),
                  pl.BlockSpec((8, 128), lambda i: (0, 0))],
        out_specs=pl.BlockSpec((2, _R, 64), lambda i: (0, i, 0)),
        out_shape=jax.ShapeDtypeStruct((2, npad, 64), jnp.float32),
    )(x, p4, g1c, wa, w2, e)


def _k6(h2, clu, zeros64):
    npad = clu.shape[0]
    rows = npad // 16
    ch = rows // 8

    @functools.partial(
        pl.kernel,
        out_type=jax.ShapeDtypeStruct((2, _MP, 64), jnp.float32),
        mesh=_mesh,
        scratch_types=[pltpu.VMEM((ch,), jnp.int32),
                       pltpu.VMEM((ch, 64), jnp.float32),
                       pltpu.VMEM_SHARED((_MP, 64), jnp.float32)],
    )
    def k(h_hbm, clu_hbm, z_hbm, out_hbm, idx_v, val_v, acc_s):
        c = lax.axis_index("c")
        s = lax.axis_index("s")

        @pl.when(s == 0)
        def _():
            pltpu.sync_copy(z_hbm, acc_s)

        plsc.subcore_barrier()
        base = s * rows
        for j in range(8):
            off = base + j * ch
            pltpu.sync_copy(clu_hbm.at[pl.ds(off, ch)], idx_v)
            pltpu.sync_copy(h_hbm.at[c, pl.ds(off, ch)], val_v)
            pltpu.sync_copy(val_v, acc_s.at[idx_v], add=True)
        plsc.subcore_barrier()

        @pl.when(s == 0)
        def _():
            pltpu.sync_copy(acc_s, out_hbm.at[c])

    return k(h2, clu, zeros64)


def _k7_body(sh_ref, aux_ref, wu0_ref, wu1_ref, wpu_ref, e_ref, z_ref):
    a = aux_ref[...]
    inv = a[:, 3:4]
    u = jnp.dot(sh_ref[0] * inv, wu0_ref[...],
                preferred_element_type=jnp.float32)
    u = u + jnp.dot(sh_ref[1] * inv, wu1_ref[...],
                    preferred_element_type=jnp.float32)
    e = e_ref[...]
    u = jnp.maximum(u + e[3:4], 0.0)
    z = jnp.dot(u, wpu_ref[...], preferred_element_type=jnp.float32)
    z_ref[...] = (z + a[:, 0:1] * e[0:1] + a[:, 1:2] * e[1:2]
                  + a[:, 2:3] * e[2:3])


def _k7(parts64, aux, wu0, wu1, wpu, e):
    return pl.pallas_call(
        _k7_body,
        grid=(16,),
        in_specs=[pl.BlockSpec((2, _BM, 64), lambda i: (0, i, 0)),
                  pl.BlockSpec((_BM, 4), lambda i: (i, 0)),
                  pl.BlockSpec((64, 128), lambda i: (0, 0)),
                  pl.BlockSpec((64, 128), lambda i: (0, 0)),
                  pl.BlockSpec((128, 128), lambda i: (0, 0)),
                  pl.BlockSpec((8, 128), lambda i: (0, 0))],
        out_specs=pl.BlockSpec((_BM, 128), lambda i: (i, 0)),
        out_shape=jax.ShapeDtypeStruct((_MP, 128), jnp.float32),
    )(parts64, aux, wu0, wu1, wpu, e)


def _k9_body(x_ref, p4_ref, zc_ref, wa_ref, w2_ref, e_ref, out_ref):
    e = e_ref[...]
    p = p4_ref[...]
    t = jnp.dot(x_ref[...], wa_ref[...], preferred_element_type=jnp.float32)
    t = t + zc_ref[...] + e[3:4]
    t = t + p[:, 0:1] * e[0:1] + p[:, 1:2] * e[1:2] + p[:, 2:3] * e[2:3]
    a = jnp.maximum(t, 0.0)
    o = jnp.dot(a, w2_ref[...], preferred_element_type=jnp.float32) + e[4:5]
    out_ref[...] = jnp.maximum(o, 0.0)


def _k9(x, p4, zc, wa, w2, e, npad):
    grid = npad // _R
    return pl.pallas_call(
        _k9_body,
        grid=(grid,),
        in_specs=[pl.BlockSpec((_R, 128), lambda i: (i, 0)),
                  pl.BlockSpec((_R, 4), lambda i: (i, 0)),
                  pl.BlockSpec((_R, 128), lambda i: (i, 0)),
                  pl.BlockSpec((128, 128), lambda i: (0, 0)),
                  pl.BlockSpec((128, 128), lambda i: (0, 0)),
                  pl.BlockSpec((8, 128), lambda i: (0, 0))],
        out_specs=pl.BlockSpec((_R, 128), lambda i: (i, 0)),
        out_shape=jax.ShapeDtypeStruct((npad, 128), jnp.float32),
    )(x, p4, zc, wa, w2, e)


def kernel(x, pos, batch, W_pre1, b_pre1, W_pre2, b_pre2,
           W_unet, b_unet, W_post1, b_post1, W_post2, b_post2):
    N = x.shape[0]
    npad = -(-N // _R) * _R

    pad3 = jnp.zeros((3, 128), jnp.float32)
    e_pre1 = jnp.concatenate([W_pre1[131:134], jnp.zeros((5, 128))], axis=0)
    e5 = jnp.concatenate([W_pre1[128:131], b_pre1[None], b_pre2[None], pad3],
                         axis=0)
    e7 = jnp.concatenate([W_post1[131:134], b_unet[None],
                          jnp.zeros((4, 128))], axis=0)
    e9 = jnp.concatenate([W_post1[128:131], b_post1[None], b_post2[None],
                          pad3], axis=0)

    clu2, p4 = _k1(pos, batch.reshape(N, 1), npad)
    clu = clu2.reshape(npad)
    parts4 = _k2(p4, clu, jnp.zeros((_MP, 4), jnp.float32))
    g1, aux = _k3(parts4, e_pre1)
    g1c = _sc_gather(g1, clu)
    h2 = _k5(x, p4, g1c, W_pre1[:128], W_pre2, e5, npad)
    parts64 = _k6(h2, clu, jnp.zeros((_MP, 64), jnp.float32))
    z = _k7(parts64, aux, W_unet[:64], W_unet[64:], W_post1[134:262], e7)
    zc = _sc_gather(z, clu)
    out = _k9(x, p4, zc, W_post1[:128], W_post2, e9, npad)
    return out[:N]


# speedup vs baseline: 2.3449x; 2.3449x over previous
import functools

import jax
import jax.numpy as jnp
from jax import lax
from jax.experimental import pallas as pl
from jax.experimental.pallas import tpu as pltpu
from jax.experimental.pallas import tpu_sc as plsc

_VOXEL = 0.1
_SIDE = 11
_NBATCH = 16
_M = _NBATCH * _SIDE ** 3
_MP = 21504
_BM = _MP // 16
_R = 1024
_NW = 32


@functools.cache
def _get_mesh():
    return plsc.VectorSubcoreMesh(core_axis_name="c", subcore_axis_name="s",
                                  num_cores=2, num_subcores=16)


def _k2(pos, batch, zeros8, npad):
    n = pos.shape[0]
    bw = (n // (_NW * 16)) * 16
    tail = n - _NW * bw
    ch = bw + tail
    n1 = bw // 16
    n2 = ch // 16
    npz = npad - n

    @functools.partial(
        pl.kernel,
        out_type=[jax.ShapeDtypeStruct((npad,), jnp.int32),
                  jax.ShapeDtypeStruct((2, _MP, 8), jnp.float32)],
        mesh=_get_mesh(),
        compiler_params=pltpu.CompilerParams(use_tc_tiling_on_sc=False,
                                             needs_layout_passes=False),
        scratch_types=[pltpu.VMEM((ch,), jnp.int32),
                       pltpu.VMEM((ch, 8), jnp.float32),
                       pltpu.VMEM((ch,), jnp.int32),
                       pltpu.VMEM((ch, 3), jnp.float32),
                       pltpu.VMEM((max(npz, 16),), jnp.int32),
                       pltpu.VMEM_SHARED((_MP, 8), jnp.float32)],
    )
    def k(pos_hbm, bat_hbm, z_hbm, clu_hbm, parts_hbm,
          idx_v, val_v, bat_v, pos_v, zt_v, acc_s):
        c = lax.axis_index("c")
        s = lax.axis_index("s")
        wid = s * 2 + c
        base = wid * bw

        @pl.when(s == 0)
        def _():
            pltpu.sync_copy(z_hbm, acc_s)

        pltpu.sync_copy(pos_hbm.at[pl.ds(base, ch)], pos_v)
        pltpu.sync_copy(bat_hbm.at[pl.ds(base, ch)], bat_v)
        pltpu.sync_copy(z_hbm.at[pl.ds(0, ch)], val_v)

        lanes = lax.iota(jnp.int32, 16)
        is_last = (jnp.zeros((16,), jnp.int32) + wid) == (_NW - 1)
        big = jnp.float32(8388608.0)
        ones16 = jnp.full((16,), 1.0, jnp.float32)

        def step(v, masked):
            r0 = v * 16
            rows = r0 + lanes
            b16 = bat_v[pl.ds(r0, 16)]
            cs = []
            m = is_last if masked else None
            for d in range(3):
                col = jnp.full((16,), d, jnp.int32)
                xd = plsc.load_gather(pos_v, [rows, col])
                plsc.store_scatter(val_v, [rows, col], xd, mask=m)
                rd = xd / jnp.float32(_VOXEL)
                cs.append(((rd + big) - big).astype(jnp.int32))
            plsc.store_scatter(val_v, [rows, jnp.full((16,), 3, jnp.int32)],
                               ones16, mask=m)
            key = ((b16 * _SIDE + cs[0]) * _SIDE + cs[1]) * _SIDE + cs[2]
            idx_v[pl.ds(r0, 16)] = key

        lax.fori_loop(0, n1, lambda v, _: (step(v, False), 0)[1], 0)
        lax.fori_loop(n1, n2, lambda v, _: (step(v, True), 0)[1], 0)

        plsc.subcore_barrier()
        pltpu.sync_copy(val_v, acc_s.at[idx_v], add=True)
        plsc.subcore_barrier()

        @pl.when(s == 0)
        def _():
            pltpu.sync_copy(acc_s, parts_hbm.at[c])

        pltpu.sync_copy(idx_v.at[pl.ds(0, bw)], clu_hbm.at[pl.ds(base, bw)])

        @pl.when(wid == _NW - 1)
        def _():
            pltpu.sync_copy(idx_v.at[pl.ds(bw, tail)],
                            clu_hbm.at[pl.ds(_NW * bw, tail)])
            for t in range(npz // 16):
                zt_v[pl.ds(16 * t, 16)] = jnp.zeros((16,), jnp.int32)
            pltpu.sync_copy(zt_v.at[pl.ds(0, npz)], clu_hbm.at[pl.ds(n, npz)])

    return k(pos, batch, zeros8)


def _k3_body(parts_ref, wc_ref, g1_ref, aux_ref):
    p = parts_ref[...]
    ssum = p[0] + p[1]
    cnt = jnp.maximum(ssum[:, 3:4], 1.0)
    inv = 1.0 / cnt
    pp = ssum[:, 0:3] * inv
    e = wc_ref[...]
    g1_ref[...] = (pp[:, 0:1] * e[0:1] + pp[:, 1:2] * e[1:2]
                   + pp[:, 2:3] * e[2:3])
    aux_ref[...] = jnp.concatenate([pp, inv], axis=1)


def _k3(parts4, wc):
    return pl.pallas_call(
        _k3_body,
        grid=(16,),
        in_specs=[pl.BlockSpec((2, _BM, 8), lambda i: (0, i, 0)),
                  pl.BlockSpec((8, 128), lambda i: (0, 0))],
        out_specs=[pl.BlockSpec((_BM, 128), lambda i: (i, 0)),
                   pl.BlockSpec((_BM, 4), lambda i: (i, 0))],
        out_shape=[jax.ShapeDtypeStruct((_MP, 128), jnp.float32),
                   jax.ShapeDtypeStruct((_MP, 4), jnp.float32)],
    )(parts4, wc)


def _sc_gather(tab, clu):
    npad = clu.shape[0]
    pw = npad // _NW
    ch = pw // 4

    @functools.partial(
        pl.kernel,
        out_type=jax.ShapeDtypeStruct((npad, 128), jnp.float32),
        mesh=_get_mesh(),
        compiler_params=pltpu.CompilerParams(use_tc_tiling_on_sc=False),
        scratch_types=[pltpu.VMEM((ch,), jnp.int32),
                       pltpu.VMEM((ch, 128), jnp.float32),
                       pltpu.SemaphoreType.DMA],
    )
    def k(tab_hbm, clu_hbm, out_hbm, idx_v, rows_v, sem):
        c = lax.axis_index("c")
        s = lax.axis_index("s")
        base = (s * 2 + c) * pw
        for j in range(4):
            off = base + j * ch
            pltpu.sync_copy(clu_hbm.at[pl.ds(off, ch)], idx_v)
            pltpu.async_copy(tab_hbm.at[idx_v], rows_v, sem).wait()
            pltpu.sync_copy(rows_v, out_hbm.at[pl.ds(off, ch)])

    return k(tab, clu)


def _k5_body(nreal, x_ref, p_ref, g_ref, wa_ref, w2_ref, e_ref, out_ref):
    i = pl.program_id(0)
    e = e_ref[...]
    p = p_ref[...]
    t = jnp.dot(x_ref[...], wa_ref[...], preferred_element_type=jnp.float32)
    t = t + g_ref[...] + e[3:4]
    t = t + p[:, 0:1] * e[0:1] + p[:, 1:2] * e[1:2] + p[:, 2:3] * e[2:3]
    a = jnp.maximum(t, 0.0)
    h = jnp.dot(a, w2_ref[...], preferred_element_type=jnp.float32) + e[4:5]
    h = jnp.maximum(h, 0.0)
    row = i * _R + lax.broadcasted_iota(jnp.int32, (_R, 1), 0)
    out_ref[...] = jnp.where(row < nreal, h, 0.0)


def _k5(x, pos, g1c, wa, w2, e, npad):
    grid = npad // _R
    return pl.pallas_call(
        functools.partial(_k5_body, x.shape[0]),
        grid=(grid,),
        in_specs=[pl.BlockSpec((_R, 128), lambda i: (i, 0)),
                  pl.BlockSpec((_R, 3), lambda i: (i, 0)),
                  pl.BlockSpec((_R, 128), lambda i: (i, 0)),
                  pl.BlockSpec((128, 128), lambda i: (0, 0)),
                  pl.BlockSpec((128, 128), lambda i: (0, 0)),
                  pl.BlockSpec((8, 128), lambda i: (0, 0))],
        out_specs=pl.BlockSpec((_R, 128), lambda i: (i, 0)),
        out_shape=jax.ShapeDtypeStruct((npad, 128), jnp.float32),
    )(x, pos, g1c, wa, w2, e)


def _k6(h, clu, zeros64):
    npad = clu.shape[0]
    rows = npad // 16
    ch = rows // 16

    @functools.partial(
        pl.kernel,
        out_type=jax.ShapeDtypeStruct((_MP, 128), jnp.float32),
        mesh=_get_mesh(),
        compiler_params=pltpu.CompilerParams(use_tc_tiling_on_sc=False),
        scratch_types=[pltpu.VMEM((ch,), jnp.int32),
                       pltpu.VMEM((ch, 64), jnp.float32),
                       pltpu.VMEM_SHARED((_MP, 64), jnp.float32)],
    )
    def k(h_hbm, clu_hbm, z_hbm, out_hbm, idx_v, val_v, acc_s):
        c = lax.axis_index("c")
        s = lax.axis_index("s")

        @pl.when(s == 0)
        def _():
            pltpu.sync_copy(z_hbm, acc_s)

        plsc.subcore_barrier()
        base = s * rows
        for j in range(16):
            off = base + j * ch
            pltpu.sync_copy(clu_hbm.at[pl.ds(off, ch)], idx_v)

            @pl.when(c == 0)
            def _():
                pltpu.sync_copy(h_hbm.at[pl.ds(off, ch), pl.ds(0, 64)], val_v)

            @pl.when(c == 1)
            def _():
                pltpu.sync_copy(h_hbm.at[pl.ds(off, ch), pl.ds(64, 64)], val_v)

            pltpu.sync_copy(val_v, acc_s.at[idx_v], add=True)
        plsc.subcore_barrier()

        @pl.when(s == 0)
        def _():
            @pl.when(c == 0)
            def _():
                pltpu.sync_copy(acc_s, out_hbm.at[:, pl.ds(0, 64)])

            @pl.when(c == 1)
            def _():
                pltpu.sync_copy(acc_s, out_hbm.at[:, pl.ds(64, 64)])

    return k(h, clu, zeros64)


def _k7_body(sh_ref, aux_ref, wu_ref, wpu_ref, e_ref, z_ref):
    a = aux_ref[...]
    inv = a[:, 3:4]
    u = jnp.dot(sh_ref[...] * inv, wu_ref[...],
                preferred_element_type=jnp.float32)
    e = e_ref[...]
    u = jnp.maximum(u + e[3:4], 0.0)
    z = jnp.dot(u, wpu_ref[...], preferred_element_type=jnp.float32)
    z_ref[...] = (z + a[:, 0:1] * e[0:1] + a[:, 1:2] * e[1:2]
                  + a[:, 2:3] * e[2:3])


def _k7(segh, aux, wu, wpu, e):
    return pl.pallas_call(
        _k7_body,
        grid=(16,),
        in_specs=[pl.BlockSpec((_BM, 128), lambda i: (i, 0)),
                  pl.BlockSpec((_BM, 4), lambda i: (i, 0)),
                  pl.BlockSpec((128, 128), lambda i: (0, 0)),
                  pl.BlockSpec((128, 128), lambda i: (0, 0)),
                  pl.BlockSpec((8, 128), lambda i: (0, 0))],
        out_specs=pl.BlockSpec((_BM, 128), lambda i: (i, 0)),
        out_shape=jax.ShapeDtypeStruct((_MP, 128), jnp.float32),
    )(segh, aux, wu, wpu, e)


def _k9_body(x_ref, p_ref, zc_ref, wa_ref, w2_ref, e_ref, out_ref):
    e = e_ref[...]
    p = p_ref[...]
    t = jnp.dot(x_ref[...], wa_ref[...], preferred_element_type=jnp.float32)
    t = t + zc_ref[...] + e[3:4]
    t = t + p[:, 0:1] * e[0:1] + p[:, 1:2] * e[1:2] + p[:, 2:3] * e[2:3]
    a = jnp.maximum(t, 0.0)
    o = jnp.dot(a, w2_ref[...], preferred_element_type=jnp.float32) + e[4:5]
    out_ref[...] = jnp.maximum(o, 0.0)


def _k9(x, pos, zc, wa, w2, e, npad):
    grid = npad // _R
    n = x.shape[0]
    return pl.pallas_call(
        _k9_body,
        grid=(grid,),
        in_specs=[pl.BlockSpec((_R, 128), lambda i: (i, 0)),
                  pl.BlockSpec((_R, 3), lambda i: (i, 0)),
                  pl.BlockSpec((_R, 128), lambda i: (i, 0)),
                  pl.BlockSpec((128, 128), lambda i: (0, 0)),
                  pl.BlockSpec((128, 128), lambda i: (0, 0)),
                  pl.BlockSpec((8, 128), lambda i: (0, 0))],
        out_specs=pl.BlockSpec((_R, 128), lambda i: (i, 0)),
        out_shape=jax.ShapeDtypeStruct((n, 128), jnp.float32),
    )(x, pos, zc, wa, w2, e)


def kernel(x, pos, batch, W_pre1, b_pre1, W_pre2, b_pre2,
           W_unet, b_unet, W_post1, b_post1, W_post2, b_post2):
    N = x.shape[0]
    npad = -(-N // _R) * _R

    pad3 = jnp.zeros((3, 128), jnp.float32)
    e_pre1 = jnp.concatenate([W_pre1[131:134], jnp.zeros((5, 128))], axis=0)
    e5 = jnp.concatenate([W_pre1[128:131], b_pre1[None], b_pre2[None], pad3],
                         axis=0)
    e7 = jnp.concatenate([W_post1[131:134], b_unet[None],
                          jnp.zeros((4, 128))], axis=0)
    e9 = jnp.concatenate([W_post1[128:131], b_post1[None], b_post2[None],
                          pad3], axis=0)

    clu, parts4 = _k2(pos, batch, jnp.zeros((_MP, 8), jnp.float32), npad)
    g1, aux = _k3(parts4, e_pre1)
    g1c = _sc_gather(g1, clu)
    h = _k5(x, pos, g1c, W_pre1[:128], W_pre2, e5, npad)
    segh = _k6(h, clu, jnp.zeros((_MP, 64), jnp.float32))
    z = _k7(segh, aux, W_unet, W_post1[134:262], e7)
    zc = _sc_gather(z, clu)
    return _k9(x, pos, zc, W_post1[:128], W_post2, e9, npad)


# speedup vs baseline: 2.5798x; 1.1002x over previous
import functools

import jax
import jax.numpy as jnp
from jax import lax
from jax.experimental import pallas as pl
from jax.experimental.pallas import tpu as pltpu
from jax.experimental.pallas import tpu_sc as plsc

_VOXEL = 0.1
_SIDE = 11
_NBATCH = 16
_M = _NBATCH * _SIDE ** 3
_MP = 21504
_BM = _MP // 16
_R = 1024
_NW = 32


@functools.cache
def _get_mesh():
    return plsc.VectorSubcoreMesh(core_axis_name="c", subcore_axis_name="s",
                                  num_cores=2, num_subcores=16)


def _k2(pos1, batch, zeros8, npad):
    n = batch.shape[0]
    bw = (n // (_NW * 16)) * 16
    tail = n - _NW * bw
    ch = bw + tail
    n1 = bw // 16
    n2 = ch // 16
    npz = npad - n

    @functools.partial(
        pl.kernel,
        out_type=[jax.ShapeDtypeStruct((npad,), jnp.int32),
                  jax.ShapeDtypeStruct((2, _MP, 8), jnp.float32)],
        mesh=_get_mesh(),
        compiler_params=pltpu.CompilerParams(use_tc_tiling_on_sc=False,
                                             needs_layout_passes=False),
        scratch_types=[pltpu.VMEM((ch,), jnp.int32),
                       pltpu.VMEM((ch, 8), jnp.float32),
                       pltpu.VMEM((ch,), jnp.int32),
                       pltpu.VMEM((ch * 3,), jnp.float32),
                       pltpu.VMEM((max(npz, 16),), jnp.int32),
                       pltpu.VMEM_SHARED((_MP, 8), jnp.float32)],
    )
    def k(pos_hbm, bat_hbm, z_hbm, clu_hbm, parts_hbm,
          idx_v, val_v, bat_v, pos_v, zt_v, acc_s):
        c = lax.axis_index("c")
        s = lax.axis_index("s")
        wid = s * 2 + c
        base = wid * bw

        @pl.when(s == 0)
        def _():
            pltpu.sync_copy(z_hbm, acc_s)

        pltpu.sync_copy(pos_hbm.at[pl.ds(base * 3, ch * 3)], pos_v)
        pltpu.sync_copy(bat_hbm.at[pl.ds(base, ch)], bat_v)
        pltpu.sync_copy(z_hbm.at[pl.ds(0, ch)], val_v)

        lanes = lax.iota(jnp.int32, 16)
        is_last = (jnp.zeros((16,), jnp.int32) + wid) == (_NW - 1)
        big = jnp.float32(8388608.0)
        ones16 = jnp.full((16,), 1.0, jnp.float32)

        def step(v, masked):
            r0 = v * 16
            rows = r0 + lanes
            b16 = bat_v[pl.ds(r0, 16)]
            rows3 = rows * 3
            cs = []
            m = is_last if masked else None
            for d in range(3):
                col = jnp.full((16,), d, jnp.int32)
                xd = plsc.load_gather(pos_v, [rows3 + d])
                plsc.store_scatter(val_v, [rows, col], xd, mask=m)
                rd = xd / jnp.float32(_VOXEL)
                cs.append(((rd + big) - big).astype(jnp.int32))
            plsc.store_scatter(val_v, [rows, jnp.full((16,), 3, jnp.int32)],
                               ones16, mask=m)
            key = ((b16 * _SIDE + cs[0]) * _SIDE + cs[1]) * _SIDE + cs[2]
            idx_v[pl.ds(r0, 16)] = key

        lax.fori_loop(0, n1, lambda v, _: (step(v, False), 0)[1], 0)
        lax.fori_loop(n1, n2, lambda v, _: (step(v, True), 0)[1], 0)

        plsc.subcore_barrier()
        pltpu.sync_copy(val_v, acc_s.at[idx_v], add=True)
        plsc.subcore_barrier()

        @pl.when(s == 0)
        def _():
            pltpu.sync_copy(acc_s, parts_hbm.at[c])

        pltpu.sync_copy(idx_v.at[pl.ds(0, bw)], clu_hbm.at[pl.ds(base, bw)])

        @pl.when(wid == _NW - 1)
        def _():
            pltpu.sync_copy(idx_v.at[pl.ds(bw, tail)],
                            clu_hbm.at[pl.ds(_NW * bw, tail)])
            for t in range(npz // 16):
                zt_v[pl.ds(16 * t, 16)] = jnp.zeros((16,), jnp.int32)
            pltpu.sync_copy(zt_v.at[pl.ds(0, npz)], clu_hbm.at[pl.ds(n, npz)])

    return k(pos1, batch, zeros8)


def _k3_body(parts_ref, wc_ref, g1_ref, aux_ref):
    p = parts_ref[...]
    ssum = p[0] + p[1]
    cnt = jnp.maximum(ssum[:, 3:4], 1.0)
    inv = 1.0 / cnt
    pp = ssum[:, 0:3] * inv
    e = wc_ref[...]
    g1_ref[...] = (pp[:, 0:1] * e[0:1] + pp[:, 1:2] * e[1:2]
                   + pp[:, 2:3] * e[2:3])
    aux_ref[...] = jnp.concatenate([pp, inv], axis=1)


def _k3(parts4, wc):
    return pl.pallas_call(
        _k3_body,
        grid=(16,),
        in_specs=[pl.BlockSpec((2, _BM, 8), lambda i: (0, i, 0)),
                  pl.BlockSpec((8, 128), lambda i: (0, 0))],
        out_specs=[pl.BlockSpec((_BM, 128), lambda i: (i, 0)),
                   pl.BlockSpec((_BM, 4), lambda i: (i, 0))],
        out_shape=[jax.ShapeDtypeStruct((_MP, 128), jnp.float32),
                   jax.ShapeDtypeStruct((_MP, 4), jnp.float32)],
    )(parts4, wc)


def _sc_gather(tab, clu):
    npad = clu.shape[0]
    pw = npad // _NW
    nch = 8
    ch = pw // nch

    @functools.partial(
        pl.kernel,
        out_type=jax.ShapeDtypeStruct((npad, 128), jnp.float32),
        mesh=_get_mesh(),
        compiler_params=pltpu.CompilerParams(use_tc_tiling_on_sc=False),
        scratch_types=[pltpu.VMEM((pw,), jnp.int32),
                       [pltpu.VMEM((ch, 128), jnp.float32)] * 2,
                       [pltpu.SemaphoreType.DMA] * 2,
                       [pltpu.SemaphoreType.DMA] * 2],
    )
    def k(tab_hbm, clu_hbm, out_hbm, idx_v, bufs, gsems, osems):
        c = lax.axis_index("c")
        s = lax.axis_index("s")
        base = (s * 2 + c) * pw
        pltpu.sync_copy(clu_hbm.at[pl.ds(base, pw)], idx_v)

        def start_gather(j):
            return pltpu.async_copy(tab_hbm.at[idx_v.at[pl.ds(j * ch, ch)]],
                                    bufs[j % 2], gsems[j % 2])

        gd = [None, None]
        od = [None, None]
        gd[0] = start_gather(0)
        for j in range(nch):
            nxt = (j + 1) % 2
            if j + 1 < nch:
                if od[nxt] is not None:
                    od[nxt].wait()
                gd[nxt] = start_gather(j + 1)
            gd[j % 2].wait()
            od[j % 2] = pltpu.async_copy(
                bufs[j % 2], out_hbm.at[pl.ds(base + j * ch, ch)],
                osems[j % 2])
        od[0].wait()
        od[1].wait()

    return k(tab, clu)


def _k5_body(nreal, x_ref, p_ref, g_ref, wa_ref, w2_ref, e_ref, out_ref):
    i = pl.program_id(0)
    e = e_ref[...]
    p = p_ref[...]
    t = jnp.dot(x_ref[...], wa_ref[...], preferred_element_type=jnp.float32)
    t = t + g_ref[...] + e[3:4]
    t = t + p[:, 0:1] * e[0:1] + p[:, 1:2] * e[1:2] + p[:, 2:3] * e[2:3]
    a = jnp.maximum(t, 0.0)
    h = jnp.dot(a, w2_ref[...], preferred_element_type=jnp.float32) + e[4:5]
    h = jnp.maximum(h, 0.0)
    row = i * _R + lax.broadcasted_iota(jnp.int32, (_R, 1), 0)
    out_ref[...] = jnp.where(row < nreal, h, 0.0)


def _k5(x, pos, g1c, wa, w2, e, npad):
    grid = npad // _R
    return pl.pallas_call(
        functools.partial(_k5_body, x.shape[0]),
        grid=(grid,),
        in_specs=[pl.BlockSpec((_R, 128), lambda i: (i, 0)),
                  pl.BlockSpec((_R, 3), lambda i: (i, 0)),
                  pl.BlockSpec((_R, 128), lambda i: (i, 0)),
                  pl.BlockSpec((128, 128), lambda i: (0, 0)),
                  pl.BlockSpec((128, 128), lambda i: (0, 0)),
                  pl.BlockSpec((8, 128), lambda i: (0, 0))],
        out_specs=pl.BlockSpec((_R, 128), lambda i: (i, 0)),
        out_shape=jax.ShapeDtypeStruct((npad, 128), jnp.float32),
    )(x, pos, g1c, wa, w2, e)


def _k6(h, clu, zeros64):
    npad = clu.shape[0]
    rows = npad // 16
    nch = 28
    ch = rows // nch

    @functools.partial(
        pl.kernel,
        out_type=jax.ShapeDtypeStruct((_MP, 128), jnp.float32),
        mesh=_get_mesh(),
        compiler_params=pltpu.CompilerParams(use_tc_tiling_on_sc=False),
        scratch_types=[pltpu.VMEM((rows,), jnp.int32),
                       [pltpu.VMEM((ch, 64), jnp.float32)] * 2,
                       [pltpu.SemaphoreType.DMA] * 2,
                       pltpu.VMEM_SHARED((_MP, 64), jnp.float32)],
    )
    def k(h_hbm, clu_hbm, z_hbm, out_hbm, idx_v, bufs, hsems, acc_s):
        c = lax.axis_index("c")
        s = lax.axis_index("s")

        @pl.when(s == 0)
        def _():
            pltpu.sync_copy(z_hbm, acc_s)

        base = s * rows
        pltpu.sync_copy(clu_hbm.at[pl.ds(base, rows)], idx_v)

        def start_load(j):
            off = base + j * ch
            b = bufs[j % 2]
            sem = hsems[j % 2]

            @pl.when(c == 0)
            def _():
                pltpu.async_copy(h_hbm.at[pl.ds(off, ch), pl.ds(0, 64)],
                                 b, sem)

            @pl.when(c == 1)
            def _():
                pltpu.async_copy(h_hbm.at[pl.ds(off, ch), pl.ds(64, 64)],
                                 b, sem)

        def drain(j):
            pltpu.make_async_copy(
                h_hbm.at[pl.ds(base, ch), pl.ds(0, 64)],
                bufs[j % 2], hsems[j % 2]).wait()

        start_load(0)
        plsc.subcore_barrier()
        for j in range(nch):
            if j + 1 < nch:
                start_load(j + 1)
            drain(j)
            pltpu.sync_copy(bufs[j % 2],
                            acc_s.at[idx_v.at[pl.ds(j * ch, ch)]], add=True)
        plsc.subcore_barrier()

        @pl.when(s == 0)
        def _():
            @pl.when(c == 0)
            def _():
                pltpu.sync_copy(acc_s, out_hbm.at[:, pl.ds(0, 64)])

            @pl.when(c == 1)
            def _():
                pltpu.sync_copy(acc_s, out_hbm.at[:, pl.ds(64, 64)])

    return k(h, clu, zeros64)


def _k7_body(sh_ref, aux_ref, wu_ref, wpu_ref, e_ref, z_ref):
    a = aux_ref[...]
    inv = a[:, 3:4]
    u = jnp.dot(sh_ref[...] * inv, wu_ref[...],
                preferred_element_type=jnp.float32)
    e = e_ref[...]
    u = jnp.maximum(u + e[3:4], 0.0)
    z = jnp.dot(u, wpu_ref[...], preferred_element_type=jnp.float32)
    z_ref[...] = (z + a[:, 0:1] * e[0:1] + a[:, 1:2] * e[1:2]
                  + a[:, 2:3] * e[2:3])


def _k7(segh, aux, wu, wpu, e):
    return pl.pallas_call(
        _k7_body,
        grid=(16,),
        in_specs=[pl.BlockSpec((_BM, 128), lambda i: (i, 0)),
                  pl.BlockSpec((_BM, 4), lambda i: (i, 0)),
                  pl.BlockSpec((128, 128), lambda i: (0, 0)),
                  pl.BlockSpec((128, 128), lambda i: (0, 0)),
                  pl.BlockSpec((8, 128), lambda i: (0, 0))],
        out_specs=pl.BlockSpec((_BM, 128), lambda i: (i, 0)),
        out_shape=jax.ShapeDtypeStruct((_MP, 128), jnp.float32),
    )(segh, aux, wu, wpu, e)


def _k9_body(x_ref, p_ref, zc_ref, wa_ref, w2_ref, e_ref, out_ref):
    e = e_ref[...]
    p = p_ref[...]
    t = jnp.dot(x_ref[...], wa_ref[...], preferred_element_type=jnp.float32)
    t = t + zc_ref[...] + e[3:4]
    t = t + p[:, 0:1] * e[0:1] + p[:, 1:2] * e[1:2] + p[:, 2:3] * e[2:3]
    a = jnp.maximum(t, 0.0)
    o = jnp.dot(a, w2_ref[...], preferred_element_type=jnp.float32) + e[4:5]
    out_ref[...] = jnp.maximum(o, 0.0)


def _k9(x, pos, zc, wa, w2, e, npad):
    grid = npad // _R
    n = x.shape[0]
    return pl.pallas_call(
        _k9_body,
        grid=(grid,),
        in_specs=[pl.BlockSpec((_R, 128), lambda i: (i, 0)),
                  pl.BlockSpec((_R, 3), lambda i: (i, 0)),
                  pl.BlockSpec((_R, 128), lambda i: (i, 0)),
                  pl.BlockSpec((128, 128), lambda i: (0, 0)),
                  pl.BlockSpec((128, 128), lambda i: (0, 0)),
                  pl.BlockSpec((8, 128), lambda i: (0, 0))],
        out_specs=pl.BlockSpec((_R, 128), lambda i: (i, 0)),
        out_shape=jax.ShapeDtypeStruct((n, 128), jnp.float32),
    )(x, pos, zc, wa, w2, e)


def kernel(x, pos, batch, W_pre1, b_pre1, W_pre2, b_pre2,
           W_unet, b_unet, W_post1, b_post1, W_post2, b_post2):
    N = x.shape[0]
    npad = -(-N // _R) * _R

    pad3 = jnp.zeros((3, 128), jnp.float32)
    e_pre1 = jnp.concatenate([W_pre1[131:134], jnp.zeros((5, 128))], axis=0)
    e5 = jnp.concatenate([W_pre1[128:131], b_pre1[None], b_pre2[None], pad3],
                         axis=0)
    e7 = jnp.concatenate([W_post1[131:134], b_unet[None],
                          jnp.zeros((4, 128))], axis=0)
    e9 = jnp.concatenate([W_post1[128:131], b_post1[None], b_post2[None],
                          pad3], axis=0)

    clu, parts4 = _k2(pos.reshape(-1), batch,
                      jnp.zeros((_MP, 8), jnp.float32), npad)
    g1, aux = _k3(parts4, e_pre1)
    g1c = _sc_gather(g1, clu)
    h = _k5(x, pos, g1c, W_pre1[:128], W_pre2, e5, npad)
    segh = _k6(h, clu, jnp.zeros((_MP, 64), jnp.float32))
    z = _k7(segh, aux, W_unet, W_post1[134:262], e7)
    zc = _sc_gather(z, clu)
    return _k9(x, pos, zc, W_post1[:128], W_post2, e9, npad)


# speedup vs baseline: 3.0229x; 1.1717x over previous
import functools

import jax
import jax.numpy as jnp
from jax import lax
from jax.experimental import pallas as pl
from jax.experimental.pallas import tpu as pltpu
from jax.experimental.pallas import tpu_sc as plsc

_VOXEL = 0.1
_SIDE = 11
_NBATCH = 16
_M = _NBATCH * _SIDE ** 3
_MP = 21504
_BM = _MP // 16
_R = 1024
_NW = 32


@functools.cache
def _get_mesh():
    return plsc.VectorSubcoreMesh(core_axis_name="c", subcore_axis_name="s",
                                  num_cores=2, num_subcores=16)


def _k2(pos1, batch, zeros8, npad):
    n = batch.shape[0]
    bw = (n // (_NW * 16)) * 16
    tail = n - _NW * bw
    ch = bw + tail
    n1 = bw // 16
    n2 = ch // 16
    npz = npad - n

    @functools.partial(
        pl.kernel,
        out_type=[jax.ShapeDtypeStruct((npad,), jnp.int32),
                  jax.ShapeDtypeStruct((2, _MP, 8), jnp.float32)],
        mesh=_get_mesh(),
        compiler_params=pltpu.CompilerParams(use_tc_tiling_on_sc=False,
                                             needs_layout_passes=False),
        scratch_types=[pltpu.VMEM((ch,), jnp.int32),
                       pltpu.VMEM((ch, 8), jnp.float32),
                       pltpu.VMEM((ch,), jnp.int32),
                       pltpu.VMEM((ch * 3,), jnp.float32),
                       pltpu.VMEM((max(npz, 16),), jnp.int32),
                       pltpu.VMEM_SHARED((_MP, 8), jnp.float32)],
    )
    def k(pos_hbm, bat_hbm, z_hbm, clu_hbm, parts_hbm,
          idx_v, val_v, bat_v, pos_v, zt_v, acc_s):
        c = lax.axis_index("c")
        s = lax.axis_index("s")
        wid = s * 2 + c
        base = wid * bw

        @pl.when(s == 0)
        def _():
            pltpu.sync_copy(z_hbm, acc_s)

        for d in range(3):
            pltpu.sync_copy(pos_hbm.at[pl.ds(d * n + base, ch)],
                            pos_v.at[pl.ds(d * ch, ch)])
        pltpu.sync_copy(bat_hbm.at[pl.ds(base, ch)], bat_v)
        pltpu.sync_copy(z_hbm.at[pl.ds(0, ch)], val_v)

        lanes = lax.iota(jnp.int32, 16)
        is_last = (jnp.zeros((16,), jnp.int32) + wid) == (_NW - 1)
        big = jnp.float32(8388608.0)
        ones16 = jnp.full((16,), 1.0, jnp.float32)

        def step(v, masked):
            r0 = v * 16
            rows = r0 + lanes
            b16 = bat_v[pl.ds(r0, 16)]
            cs = []
            m = is_last if masked else None
            for d in range(3):
                col = jnp.full((16,), d, jnp.int32)
                xd = pos_v[pl.ds(d * ch + r0, 16)]
                plsc.store_scatter(val_v, [rows, col], xd, mask=m)
                rd = xd / jnp.float32(_VOXEL)
                cs.append(((rd + big) - big).astype(jnp.int32))
            plsc.store_scatter(val_v, [rows, jnp.full((16,), 3, jnp.int32)],
                               ones16, mask=m)
            key = ((b16 * _SIDE + cs[0]) * _SIDE + cs[1]) * _SIDE + cs[2]
            idx_v[pl.ds(r0, 16)] = key

        lax.fori_loop(0, n1, lambda v, _: (step(v, False), 0)[1], 0)
        lax.fori_loop(n1, n2, lambda v, _: (step(v, True), 0)[1], 0)

        plsc.subcore_barrier()
        pltpu.sync_copy(val_v, acc_s.at[idx_v], add=True)
        plsc.subcore_barrier()

        @pl.when(s == 0)
        def _():
            pltpu.sync_copy(acc_s, parts_hbm.at[c])

        pltpu.sync_copy(idx_v.at[pl.ds(0, bw)], clu_hbm.at[pl.ds(base, bw)])

        @pl.when(wid == _NW - 1)
        def _():
            pltpu.sync_copy(idx_v.at[pl.ds(bw, tail)],
                            clu_hbm.at[pl.ds(_NW * bw, tail)])
            for t in range(npz // 16):
                zt_v[pl.ds(16 * t, 16)] = jnp.zeros((16,), jnp.int32)
            pltpu.sync_copy(zt_v.at[pl.ds(0, npz)], clu_hbm.at[pl.ds(n, npz)])

    return k(pos1, batch, zeros8)


def _k3_body(parts_ref, wc_ref, g1_ref, aux_ref):
    p = parts_ref[...]
    ssum = p[0] + p[1]
    cnt = jnp.maximum(ssum[:, 3:4], 1.0)
    inv = 1.0 / cnt
    pp = ssum[:, 0:3] * inv
    e = wc_ref[...]
    g1_ref[...] = (pp[:, 0:1] * e[0:1] + pp[:, 1:2] * e[1:2]
                   + pp[:, 2:3] * e[2:3])
    aux_ref[...] = jnp.concatenate([pp, inv], axis=1)


def _k3(parts4, wc):
    return pl.pallas_call(
        _k3_body,
        grid=(16,),
        in_specs=[pl.BlockSpec((2, _BM, 8), lambda i: (0, i, 0)),
                  pl.BlockSpec((8, 128), lambda i: (0, 0))],
        out_specs=[pl.BlockSpec((_BM, 128), lambda i: (i, 0)),
                   pl.BlockSpec((_BM, 4), lambda i: (i, 0))],
        out_shape=[jax.ShapeDtypeStruct((_MP, 128), jnp.float32),
                   jax.ShapeDtypeStruct((_MP, 4), jnp.float32)],
    )(parts4, wc)


def _sc_gather(tab, clu):
    npad = clu.shape[0]
    pw = npad // _NW
    nch = 8
    ch = pw // nch

    @functools.partial(
        pl.kernel,
        out_type=jax.ShapeDtypeStruct((npad, 128), jnp.float32),
        mesh=_get_mesh(),
        compiler_params=pltpu.CompilerParams(use_tc_tiling_on_sc=False),
        scratch_types=[pltpu.VMEM((pw,), jnp.int32),
                       [pltpu.VMEM((ch, 128), jnp.float32)] * 2,
                       [pltpu.SemaphoreType.DMA] * 2,
                       [pltpu.SemaphoreType.DMA] * 2],
    )
    def k(tab_hbm, clu_hbm, out_hbm, idx_v, bufs, gsems, osems):
        c = lax.axis_index("c")
        s = lax.axis_index("s")
        base = (s * 2 + c) * pw
        pltpu.sync_copy(clu_hbm.at[pl.ds(base, pw)], idx_v)

        def start_gather(j):
            return pltpu.async_copy(tab_hbm.at[idx_v.at[pl.ds(j * ch, ch)]],
                                    bufs[j % 2], gsems[j % 2])

        gd = [None, None]
        od = [None, None]
        gd[0] = start_gather(0)
        for j in range(nch):
            nxt = (j + 1) % 2
            if j + 1 < nch:
                if od[nxt] is not None:
                    od[nxt].wait()
                gd[nxt] = start_gather(j + 1)
            gd[j % 2].wait()
            od[j % 2] = pltpu.async_copy(
                bufs[j % 2], out_hbm.at[pl.ds(base + j * ch, ch)],
                osems[j % 2])
        od[0].wait()
        od[1].wait()

    return k(tab, clu)


def _k5_body(nreal, x_ref, p_ref, g_ref, wa_ref, w2_ref, e_ref, out_ref):
    i = pl.program_id(0)
    e = e_ref[...]
    pt = p_ref[...]
    t = jnp.dot(x_ref[...], wa_ref[...], preferred_element_type=jnp.float32)
    t = t + g_ref[...] + e[3:4]
    t = t + lax.dot_general(pt, e_ref[...],
                            (((0,), (0,)), ((), ())),
                            preferred_element_type=jnp.float32)
    a = jnp.maximum(t, 0.0)
    h = jnp.dot(a, w2_ref[...], preferred_element_type=jnp.float32) + e[4:5]
    h = jnp.maximum(h, 0.0)
    row = i * _R + lax.broadcasted_iota(jnp.int32, (_R, 1), 0)
    out_ref[...] = jnp.where(row < nreal, h, 0.0)


def _k5(x, pos, g1c, wa, w2, e, npad):
    grid = npad // _R
    return pl.pallas_call(
        functools.partial(_k5_body, x.shape[0]),
        grid=(grid,),
        in_specs=[pl.BlockSpec((_R, 128), lambda i: (i, 0)),
                  pl.BlockSpec((8, _R), lambda i: (0, i)),
                  pl.BlockSpec((_R, 128), lambda i: (i, 0)),
                  pl.BlockSpec((128, 128), lambda i: (0, 0)),
                  pl.BlockSpec((128, 128), lambda i: (0, 0)),
                  pl.BlockSpec((8, 128), lambda i: (0, 0))],
        out_specs=pl.BlockSpec((_R, 128), lambda i: (i, 0)),
        out_shape=jax.ShapeDtypeStruct((npad, 128), jnp.float32),
    )(x, pos, g1c, wa, w2, e)


def _k6(h, clu, zeros64):
    npad = clu.shape[0]
    rows = npad // 16
    nch = 28
    ch = rows // nch

    @functools.partial(
        pl.kernel,
        out_type=jax.ShapeDtypeStruct((_MP, 128), jnp.float32),
        mesh=_get_mesh(),
        compiler_params=pltpu.CompilerParams(use_tc_tiling_on_sc=False),
        scratch_types=[pltpu.VMEM((rows,), jnp.int32),
                       [pltpu.VMEM((ch, 64), jnp.float32)] * 2,
                       [pltpu.SemaphoreType.DMA] * 2,
                       pltpu.VMEM_SHARED((_MP, 64), jnp.float32)],
    )
    def k(h_hbm, clu_hbm, z_hbm, out_hbm, idx_v, bufs, hsems, acc_s):
        c = lax.axis_index("c")
        s = lax.axis_index("s")

        @pl.when(s == 0)
        def _():
            pltpu.sync_copy(z_hbm, acc_s)

        base = s * rows
        pltpu.sync_copy(clu_hbm.at[pl.ds(base, rows)], idx_v)

        def start_load(j):
            off = base + j * ch
            b = bufs[j % 2]
            sem = hsems[j % 2]

            @pl.when(c == 0)
            def _():
                pltpu.async_copy(h_hbm.at[pl.ds(off, ch), pl.ds(0, 64)],
                                 b, sem)

            @pl.when(c == 1)
            def _():
                pltpu.async_copy(h_hbm.at[pl.ds(off, ch), pl.ds(64, 64)],
                                 b, sem)

        def drain(j):
            pltpu.make_async_copy(
                h_hbm.at[pl.ds(base, ch), pl.ds(0, 64)],
                bufs[j % 2], hsems[j % 2]).wait()

        start_load(0)
        plsc.subcore_barrier()
        for j in range(nch):
            if j + 1 < nch:
                start_load(j + 1)
            drain(j)
            pltpu.sync_copy(bufs[j % 2],
                            acc_s.at[idx_v.at[pl.ds(j * ch, ch)]], add=True)
        plsc.subcore_barrier()

        @pl.when(s == 0)
        def _():
            @pl.when(c == 0)
            def _():
                pltpu.sync_copy(acc_s, out_hbm.at[:, pl.ds(0, 64)])

            @pl.when(c == 1)
            def _():
                pltpu.sync_copy(acc_s, out_hbm.at[:, pl.ds(64, 64)])

    return k(h, clu, zeros64)


def _k7_body(sh_ref, aux_ref, wu_ref, wpu_ref, e_ref, z_ref):
    a = aux_ref[...]
    inv = a[:, 3:4]
    u = jnp.dot(sh_ref[...] * inv, wu_ref[...],
                preferred_element_type=jnp.float32)
    e = e_ref[...]
    u = jnp.maximum(u + e[3:4], 0.0)
    z = jnp.dot(u, wpu_ref[...], preferred_element_type=jnp.float32)
    z_ref[...] = (z + a[:, 0:1] * e[0:1] + a[:, 1:2] * e[1:2]
                  + a[:, 2:3] * e[2:3])


def _k7(segh, aux, wu, wpu, e):
    return pl.pallas_call(
        _k7_body,
        grid=(16,),
        in_specs=[pl.BlockSpec((_BM, 128), lambda i: (i, 0)),
                  pl.BlockSpec((_BM, 4), lambda i: (i, 0)),
                  pl.BlockSpec((128, 128), lambda i: (0, 0)),
                  pl.BlockSpec((128, 128), lambda i: (0, 0)),
                  pl.BlockSpec((8, 128), lambda i: (0, 0))],
        out_specs=pl.BlockSpec((_BM, 128), lambda i: (i, 0)),
        out_shape=jax.ShapeDtypeStruct((_MP, 128), jnp.float32),
    )(segh, aux, wu, wpu, e)


def _k9_body(x_ref, p_ref, zc_ref, wa_ref, w2_ref, e_ref, out_ref):
    e = e_ref[...]
    pt = p_ref[...]
    t = jnp.dot(x_ref[...], wa_ref[...], preferred_element_type=jnp.float32)
    t = t + zc_ref[...] + e[3:4]
    t = t + lax.dot_general(pt, e_ref[...],
                            (((0,), (0,)), ((), ())),
                            preferred_element_type=jnp.float32)
    a = jnp.maximum(t, 0.0)
    o = jnp.dot(a, w2_ref[...], preferred_element_type=jnp.float32) + e[4:5]
    out_ref[...] = jnp.maximum(o, 0.0)


def _k9(x, pos, zc, wa, w2, e, npad):
    grid = npad // _R
    n = x.shape[0]
    return pl.pallas_call(
        _k9_body,
        grid=(grid,),
        in_specs=[pl.BlockSpec((_R, 128), lambda i: (i, 0)),
                  pl.BlockSpec((8, _R), lambda i: (0, i)),
                  pl.BlockSpec((_R, 128), lambda i: (i, 0)),
                  pl.BlockSpec((128, 128), lambda i: (0, 0)),
                  pl.BlockSpec((128, 128), lambda i: (0, 0)),
                  pl.BlockSpec((8, 128), lambda i: (0, 0))],
        out_specs=pl.BlockSpec((_R, 128), lambda i: (i, 0)),
        out_shape=jax.ShapeDtypeStruct((n, 128), jnp.float32),
    )(x, pos, zc, wa, w2, e)


def kernel(x, pos, batch, W_pre1, b_pre1, W_pre2, b_pre2,
           W_unet, b_unet, W_post1, b_post1, W_post2, b_post2):
    N = x.shape[0]
    npad = -(-N // _R) * _R

    pad3 = jnp.zeros((3, 128), jnp.float32)
    e_pre1 = jnp.concatenate([W_pre1[131:134], jnp.zeros((5, 128))], axis=0)
    e5 = jnp.concatenate([W_pre1[128:131], b_pre1[None], b_pre2[None], pad3],
                         axis=0)
    e7 = jnp.concatenate([W_post1[131:134], b_unet[None],
                          jnp.zeros((4, 128))], axis=0)
    e9 = jnp.concatenate([W_post1[128:131], b_post1[None], b_post2[None],
                          pad3], axis=0)

    pos_t8 = jnp.zeros((8, N), jnp.float32).at[:3].set(pos.T)
    clu, parts4 = _k2(pos_t8.reshape(-1), batch,
                      jnp.zeros((_MP, 8), jnp.float32), npad)
    g1, aux = _k3(parts4, e_pre1)
    g1c = _sc_gather(g1, clu)
    h = _k5(x, pos_t8, g1c, W_pre1[:128], W_pre2, e5, npad)
    segh = _k6(h, clu, jnp.zeros((_MP, 64), jnp.float32))
    z = _k7(segh, aux, W_unet, W_post1[134:262], e7)
    zc = _sc_gather(z, clu)
    return _k9(x, pos_t8, zc, W_post1[:128], W_post2, e9, npad)


# speedup vs baseline: 3.4157x; 1.1299x over previous
import functools

import jax
import jax.numpy as jnp
from jax import lax
from jax.experimental import pallas as pl
from jax.experimental.pallas import tpu as pltpu
from jax.experimental.pallas import tpu_sc as plsc

_VOXEL = 0.1
_SIDE = 11
_NBATCH = 16
_M = _NBATCH * _SIDE ** 3
_MP = 21504
_BM = _MP // 16
_R = 1024
_NW = 32


@functools.cache
def _get_mesh():
    return plsc.VectorSubcoreMesh(core_axis_name="c", subcore_axis_name="s",
                                  num_cores=2, num_subcores=16)


def _k2(pos1, batch, zeros8, npad):
    n = batch.shape[0]
    bw = (n // (_NW * 16)) * 16
    tail = n - _NW * bw
    ch = bw + tail
    n1 = bw // 16
    n2 = ch // 16
    npz = npad - n

    @functools.partial(
        pl.kernel,
        out_type=[jax.ShapeDtypeStruct((npad,), jnp.int32),
                  jax.ShapeDtypeStruct((2, _MP, 8), jnp.float32)],
        mesh=_get_mesh(),
        compiler_params=pltpu.CompilerParams(use_tc_tiling_on_sc=False,
                                             needs_layout_passes=False),
        scratch_types=[pltpu.VMEM((ch,), jnp.int32),
                       pltpu.VMEM((ch, 8), jnp.float32),
                       pltpu.VMEM((ch,), jnp.int32),
                       pltpu.VMEM((ch * 3,), jnp.float32),
                       pltpu.VMEM((max(npz, 16),), jnp.int32),
                       pltpu.VMEM_SHARED((_MP, 8), jnp.float32)],
    )
    def k(pos_hbm, bat_hbm, z_hbm, clu_hbm, parts_hbm,
          idx_v, val_v, bat_v, pos_v, zt_v, acc_s):
        c = lax.axis_index("c")
        s = lax.axis_index("s")
        wid = s * 2 + c
        base = wid * bw

        @pl.when(s == 0)
        def _():
            pltpu.sync_copy(z_hbm, acc_s)

        for d in range(3):
            pltpu.sync_copy(pos_hbm.at[pl.ds(d * n + base, ch)],
                            pos_v.at[pl.ds(d * ch, ch)])
        pltpu.sync_copy(bat_hbm.at[pl.ds(base, ch)], bat_v)
        pltpu.sync_copy(z_hbm.at[pl.ds(0, ch)], val_v)

        lanes = lax.iota(jnp.int32, 16)
        is_last = (jnp.zeros((16,), jnp.int32) + wid) == (_NW - 1)
        big = jnp.float32(8388608.0)
        ones16 = jnp.full((16,), 1.0, jnp.float32)

        def step(v, masked):
            r0 = v * 16
            rows = r0 + lanes
            b16 = bat_v[pl.ds(r0, 16)]
            cs = []
            m = is_last if masked else None
            for d in range(3):
                col = jnp.full((16,), d, jnp.int32)
                xd = pos_v[pl.ds(d * ch + r0, 16)]
                plsc.store_scatter(val_v, [rows, col], xd, mask=m)
                rd = xd / jnp.float32(_VOXEL)
                cs.append(((rd + big) - big).astype(jnp.int32))
            plsc.store_scatter(val_v, [rows, jnp.full((16,), 3, jnp.int32)],
                               ones16, mask=m)
            key = ((b16 * _SIDE + cs[0]) * _SIDE + cs[1]) * _SIDE + cs[2]
            idx_v[pl.ds(r0, 16)] = key

        lax.fori_loop(0, n1, lambda v, _: (step(v, False), 0)[1], 0)
        lax.fori_loop(n1, n2, lambda v, _: (step(v, True), 0)[1], 0)

        plsc.subcore_barrier()
        pltpu.sync_copy(val_v, acc_s.at[idx_v], add=True)
        plsc.subcore_barrier()

        @pl.when(s == 0)
        def _():
            pltpu.sync_copy(acc_s, parts_hbm.at[c])

        pltpu.sync_copy(idx_v.at[pl.ds(0, bw)], clu_hbm.at[pl.ds(base, bw)])

        @pl.when(wid == _NW - 1)
        def _():
            pltpu.sync_copy(idx_v.at[pl.ds(bw, tail)],
                            clu_hbm.at[pl.ds(_NW * bw, tail)])
            for t in range(npz // 16):
                zt_v[pl.ds(16 * t, 16)] = jnp.zeros((16,), jnp.int32)
            pltpu.sync_copy(zt_v.at[pl.ds(0, npz)], clu_hbm.at[pl.ds(n, npz)])

    return k(pos1, batch, zeros8)


def _k3_body(parts_ref, wc_ref, g1_ref, aux_ref):
    p = parts_ref[...]
    ssum = p[0] + p[1]
    cnt = jnp.maximum(ssum[:, 3:4], 1.0)
    inv = 1.0 / cnt
    pp = ssum[:, 0:3] * inv
    e = wc_ref[...]
    g1_ref[...] = (pp[:, 0:1] * e[0:1] + pp[:, 1:2] * e[1:2]
                   + pp[:, 2:3] * e[2:3])
    aux_ref[...] = jnp.concatenate([pp, inv], axis=1)


def _k3(parts4, wc):
    return pl.pallas_call(
        _k3_body,
        grid=(16,),
        in_specs=[pl.BlockSpec((2, _BM, 8), lambda i: (0, i, 0)),
                  pl.BlockSpec((8, 128), lambda i: (0, 0))],
        out_specs=[pl.BlockSpec((_BM, 128), lambda i: (i, 0)),
                   pl.BlockSpec((_BM, 4), lambda i: (i, 0))],
        out_shape=[jax.ShapeDtypeStruct((_MP, 128), jnp.float32),
                   jax.ShapeDtypeStruct((_MP, 4), jnp.float32)],
    )(parts4, wc)


def _sc_gather(tab, clu):
    npad = clu.shape[0]
    rows = npad // 16
    nch = 28
    ch = rows // nch
    mrows = _MP // 16

    @functools.partial(
        pl.kernel,
        out_type=jax.ShapeDtypeStruct((npad, 128), jnp.float32),
        mesh=_get_mesh(),
        compiler_params=pltpu.CompilerParams(use_tc_tiling_on_sc=False),
        scratch_types=[pltpu.VMEM((rows,), jnp.int32),
                       [pltpu.VMEM((ch, 64), jnp.float32)] * 2,
                       [pltpu.SemaphoreType.DMA] * 2,
                       [pltpu.SemaphoreType.DMA] * 2,
                       pltpu.VMEM_SHARED((_MP, 64), jnp.float32)],
    )
    def k(tab_hbm, clu_hbm, out_hbm, idx_v, bufs, gsems, osems, tab_s):
        c = lax.axis_index("c")
        s = lax.axis_index("s")
        base = s * rows
        pltpu.sync_copy(clu_hbm.at[pl.ds(base, rows)], idx_v)
        mb = s * mrows

        @pl.when(c == 0)
        def _():
            pltpu.sync_copy(tab_hbm.at[pl.ds(mb, mrows), pl.ds(0, 64)],
                            tab_s.at[pl.ds(mb, mrows)])

        @pl.when(c == 1)
        def _():
            pltpu.sync_copy(tab_hbm.at[pl.ds(mb, mrows), pl.ds(64, 64)],
                            tab_s.at[pl.ds(mb, mrows)])

        plsc.subcore_barrier()

        def start_gather(j):
            return pltpu.async_copy(
                tab_s.at[idx_v.at[pl.ds(j * ch, ch)]],
                bufs[j % 2], gsems[j % 2])

        def start_out(j):
            b = bufs[j % 2]
            sem = osems[j % 2]
            off = base + j * ch

            @pl.when(c == 0)
            def _():
                pltpu.async_copy(b, out_hbm.at[pl.ds(off, ch), pl.ds(0, 64)],
                                 sem)

            @pl.when(c == 1)
            def _():
                pltpu.async_copy(b, out_hbm.at[pl.ds(off, ch), pl.ds(64, 64)],
                                 sem)

        def drain_out(j):
            pltpu.make_async_copy(
                out_hbm.at[pl.ds(base, ch), pl.ds(0, 64)],
                bufs[j % 2], osems[j % 2]).wait()

        gd = [None, None]
        started = [False, False]
        gd[0] = start_gather(0)
        for j in range(nch):
            nxt = (j + 1) % 2
            if j + 1 < nch:
                if started[nxt]:
                    drain_out(j - 1)
                gd[nxt] = start_gather(j + 1)
            gd[j % 2].wait()
            start_out(j)
            started[j % 2] = True
        drain_out(nch - 2)
        drain_out(nch - 1)

    return k(tab, clu)


def _k5_body(nreal, x_ref, p_ref, g_ref, wa_ref, w2_ref, e_ref, out_ref):
    i = pl.program_id(0)
    e = e_ref[...]
    pt = p_ref[...]
    t = jnp.dot(x_ref[...], wa_ref[...], preferred_element_type=jnp.float32)
    t = t + g_ref[...] + e[3:4]
    t = t + lax.dot_general(pt, e_ref[...],
                            (((0,), (0,)), ((), ())),
                            preferred_element_type=jnp.float32)
    a = jnp.maximum(t, 0.0)
    h = jnp.dot(a, w2_ref[...], preferred_element_type=jnp.float32) + e[4:5]
    h = jnp.maximum(h, 0.0)
    row = i * _R + lax.broadcasted_iota(jnp.int32, (_R, 1), 0)
    out_ref[...] = jnp.where(row < nreal, h, 0.0)


def _k5(x, pos, g1c, wa, w2, e, npad):
    grid = npad // _R
    return pl.pallas_call(
        functools.partial(_k5_body, x.shape[0]),
        grid=(grid,),
        in_specs=[pl.BlockSpec((_R, 128), lambda i: (i, 0)),
                  pl.BlockSpec((8, _R), lambda i: (0, i)),
                  pl.BlockSpec((_R, 128), lambda i: (i, 0)),
                  pl.BlockSpec((128, 128), lambda i: (0, 0)),
                  pl.BlockSpec((128, 128), lambda i: (0, 0)),
                  pl.BlockSpec((8, 128), lambda i: (0, 0))],
        out_specs=pl.BlockSpec((_R, 128), lambda i: (i, 0)),
        out_shape=jax.ShapeDtypeStruct((npad, 128), jnp.float32),
    )(x, pos, g1c, wa, w2, e)


def _k6(h, clu, zeros64):
    npad = clu.shape[0]
    rows = npad // 16
    nch = 28
    ch = rows // nch

    @functools.partial(
        pl.kernel,
        out_type=jax.ShapeDtypeStruct((_MP, 128), jnp.float32),
        mesh=_get_mesh(),
        compiler_params=pltpu.CompilerParams(use_tc_tiling_on_sc=False),
        scratch_types=[pltpu.VMEM((rows,), jnp.int32),
                       [pltpu.VMEM((ch, 64), jnp.float32)] * 2,
                       [pltpu.SemaphoreType.DMA] * 2,
                       pltpu.VMEM_SHARED((_MP, 64), jnp.float32)],
    )
    def k(h_hbm, clu_hbm, z_hbm, out_hbm, idx_v, bufs, hsems, acc_s):
        c = lax.axis_index("c")
        s = lax.axis_index("s")

        @pl.when(s == 0)
        def _():
            pltpu.sync_copy(z_hbm, acc_s)

        base = s * rows
        pltpu.sync_copy(clu_hbm.at[pl.ds(base, rows)], idx_v)

        def start_load(j):
            off = base + j * ch
            b = bufs[j % 2]
            sem = hsems[j % 2]

            @pl.when(c == 0)
            def _():
                pltpu.async_copy(h_hbm.at[pl.ds(off, ch), pl.ds(0, 64)],
                                 b, sem)

            @pl.when(c == 1)
            def _():
                pltpu.async_copy(h_hbm.at[pl.ds(off, ch), pl.ds(64, 64)],
                                 b, sem)

        def drain(j):
            pltpu.make_async_copy(
                h_hbm.at[pl.ds(base, ch), pl.ds(0, 64)],
                bufs[j % 2], hsems[j % 2]).wait()

        start_load(0)
        plsc.subcore_barrier()
        for j in range(nch):
            if j + 1 < nch:
                start_load(j + 1)
            drain(j)
            pltpu.sync_copy(bufs[j % 2],
                            acc_s.at[idx_v.at[pl.ds(j * ch, ch)]], add=True)
        plsc.subcore_barrier()

        @pl.when(s == 0)
        def _():
            @pl.when(c == 0)
            def _():
                pltpu.sync_copy(acc_s, out_hbm.at[:, pl.ds(0, 64)])

            @pl.when(c == 1)
            def _():
                pltpu.sync_copy(acc_s, out_hbm.at[:, pl.ds(64, 64)])

    return k(h, clu, zeros64)


def _k7_body(sh_ref, aux_ref, wu_ref, wpu_ref, e_ref, z_ref):
    a = aux_ref[...]
    inv = a[:, 3:4]
    u = jnp.dot(sh_ref[...] * inv, wu_ref[...],
                preferred_element_type=jnp.float32)
    e = e_ref[...]
    u = jnp.maximum(u + e[3:4], 0.0)
    z = jnp.dot(u, wpu_ref[...], preferred_element_type=jnp.float32)
    z_ref[...] = (z + a[:, 0:1] * e[0:1] + a[:, 1:2] * e[1:2]
                  + a[:, 2:3] * e[2:3])


def _k7(segh, aux, wu, wpu, e):
    return pl.pallas_call(
        _k7_body,
        grid=(16,),
        in_specs=[pl.BlockSpec((_BM, 128), lambda i: (i, 0)),
                  pl.BlockSpec((_BM, 4), lambda i: (i, 0)),
                  pl.BlockSpec((128, 128), lambda i: (0, 0)),
                  pl.BlockSpec((128, 128), lambda i: (0, 0)),
                  pl.BlockSpec((8, 128), lambda i: (0, 0))],
        out_specs=pl.BlockSpec((_BM, 128), lambda i: (i, 0)),
        out_shape=jax.ShapeDtypeStruct((_MP, 128), jnp.float32),
    )(segh, aux, wu, wpu, e)


def _k9_body(x_ref, p_ref, zc_ref, wa_ref, w2_ref, e_ref, out_ref):
    e = e_ref[...]
    pt = p_ref[...]
    t = jnp.dot(x_ref[...], wa_ref[...], preferred_element_type=jnp.float32)
    t = t + zc_ref[...] + e[3:4]
    t = t + lax.dot_general(pt, e_ref[...],
                            (((0,), (0,)), ((), ())),
                            preferred_element_type=jnp.float32)
    a = jnp.maximum(t, 0.0)
    o = jnp.dot(a, w2_ref[...], preferred_element_type=jnp.float32) + e[4:5]
    out_ref[...] = jnp.maximum(o, 0.0)


def _k9(x, pos, zc, wa, w2, e, npad):
    grid = npad // _R
    n = x.shape[0]
    return pl.pallas_call(
        _k9_body,
        grid=(grid,),
        in_specs=[pl.BlockSpec((_R, 128), lambda i: (i, 0)),
                  pl.BlockSpec((8, _R), lambda i: (0, i)),
                  pl.BlockSpec((_R, 128), lambda i: (i, 0)),
                  pl.BlockSpec((128, 128), lambda i: (0, 0)),
                  pl.BlockSpec((128, 128), lambda i: (0, 0)),
                  pl.BlockSpec((8, 128), lambda i: (0, 0))],
        out_specs=pl.BlockSpec((_R, 128), lambda i: (i, 0)),
        out_shape=jax.ShapeDtypeStruct((n, 128), jnp.float32),
    )(x, pos, zc, wa, w2, e)


def kernel(x, pos, batch, W_pre1, b_pre1, W_pre2, b_pre2,
           W_unet, b_unet, W_post1, b_post1, W_post2, b_post2):
    N = x.shape[0]
    npad = -(-N // _R) * _R

    pad3 = jnp.zeros((3, 128), jnp.float32)
    e_pre1 = jnp.concatenate([W_pre1[131:134], jnp.zeros((5, 128))], axis=0)
    e5 = jnp.concatenate([W_pre1[128:131], b_pre1[None], b_pre2[None], pad3],
                         axis=0)
    e7 = jnp.concatenate([W_post1[131:134], b_unet[None],
                          jnp.zeros((4, 128))], axis=0)
    e9 = jnp.concatenate([W_post1[128:131], b_post1[None], b_post2[None],
                          pad3], axis=0)

    pos_t8 = jnp.zeros((8, N), jnp.float32).at[:3].set(pos.T)
    clu, parts4 = _k2(pos_t8.reshape(-1), batch,
                      jnp.zeros((_MP, 8), jnp.float32), npad)
    g1, aux = _k3(parts4, e_pre1)
    g1c = _sc_gather(g1, clu)
    h = _k5(x, pos_t8, g1c, W_pre1[:128], W_pre2, e5, npad)
    segh = _k6(h, clu, jnp.zeros((_MP, 64), jnp.float32))
    z = _k7(segh, aux, W_unet, W_post1[134:262], e7)
    zc = _sc_gather(z, clu)
    return _k9(x, pos_t8, zc, W_post1[:128], W_post2, e9, npad)


# speedup vs baseline: 3.9436x; 1.1546x over previous
import functools

import jax
import jax.numpy as jnp
from jax import lax
from jax.experimental import pallas as pl
from jax.experimental.pallas import tpu as pltpu
from jax.experimental.pallas import tpu_sc as plsc

_VOXEL = 0.1
_SIDE = 11
_NBATCH = 16
_M = _NBATCH * _SIDE ** 3
_MP = 21504
_BM = _MP // 16
_R = 2048
_NW = 32


@functools.cache
def _get_mesh():
    return plsc.VectorSubcoreMesh(core_axis_name="c", subcore_axis_name="s",
                                  num_cores=2, num_subcores=16)


def _k2(pos1, batch, zeros8, npad):
    n = batch.shape[0]
    bw = (n // (_NW * 16)) * 16
    tail = n - _NW * bw
    ch = bw + tail
    n1 = bw // 16
    n2 = ch // 16
    npz = npad - n

    @functools.partial(
        pl.kernel,
        out_type=[jax.ShapeDtypeStruct((npad,), jnp.int32),
                  jax.ShapeDtypeStruct((2, _MP, 8), jnp.float32)],
        mesh=_get_mesh(),
        compiler_params=pltpu.CompilerParams(use_tc_tiling_on_sc=False,
                                             needs_layout_passes=False),
        scratch_types=[pltpu.VMEM((ch,), jnp.int32),
                       pltpu.VMEM((ch, 8), jnp.float32),
                       pltpu.VMEM((ch,), jnp.int32),
                       pltpu.VMEM((ch * 3,), jnp.float32),
                       pltpu.VMEM((max(npz, 16),), jnp.int32),
                       pltpu.VMEM_SHARED((_MP, 8), jnp.float32)],
    )
    def k(pos_hbm, bat_hbm, z_hbm, clu_hbm, parts_hbm,
          idx_v, val_v, bat_v, pos_v, zt_v, acc_s):
        c = lax.axis_index("c")
        s = lax.axis_index("s")
        wid = s * 2 + c
        base = wid * bw

        @pl.when(s == 0)
        def _():
            pltpu.sync_copy(z_hbm, acc_s)

        for d in range(3):
            pltpu.sync_copy(pos_hbm.at[pl.ds(d * n + base, ch)],
                            pos_v.at[pl.ds(d * ch, ch)])
        pltpu.sync_copy(bat_hbm.at[pl.ds(base, ch)], bat_v)
        pltpu.sync_copy(z_hbm.at[pl.ds(0, ch)], val_v)

        lanes = lax.iota(jnp.int32, 16)
        is_last = (jnp.zeros((16,), jnp.int32) + wid) == (_NW - 1)
        big = jnp.float32(8388608.0)
        ones16 = jnp.full((16,), 1.0, jnp.float32)

        def step(v, masked):
            r0 = v * 16
            rows = r0 + lanes
            b16 = bat_v[pl.ds(r0, 16)]
            cs = []
            m = is_last if masked else None
            for d in range(3):
                col = jnp.full((16,), d, jnp.int32)
                xd = pos_v[pl.ds(d * ch + r0, 16)]
                plsc.store_scatter(val_v, [rows, col], xd, mask=m)
                rd = xd / jnp.float32(_VOXEL)
                cs.append(((rd + big) - big).astype(jnp.int32))
            plsc.store_scatter(val_v, [rows, jnp.full((16,), 3, jnp.int32)],
                               ones16, mask=m)
            key = ((b16 * _SIDE + cs[0]) * _SIDE + cs[1]) * _SIDE + cs[2]
            idx_v[pl.ds(r0, 16)] = key

        lax.fori_loop(0, n1, lambda v, _: (step(v, False), 0)[1], 0)
        lax.fori_loop(n1, n2, lambda v, _: (step(v, True), 0)[1], 0)

        plsc.subcore_barrier()
        pltpu.sync_copy(val_v, acc_s.at[idx_v], add=True)
        plsc.subcore_barrier()

        @pl.when(s == 0)
        def _():
            pltpu.sync_copy(acc_s, parts_hbm.at[c])

        pltpu.sync_copy(idx_v.at[pl.ds(0, bw)], clu_hbm.at[pl.ds(base, bw)])

        @pl.when(wid == _NW - 1)
        def _():
            pltpu.sync_copy(idx_v.at[pl.ds(bw, tail)],
                            clu_hbm.at[pl.ds(_NW * bw, tail)])
            for t in range(npz // 16):
                zt_v[pl.ds(16 * t, 16)] = jnp.zeros((16,), jnp.int32)
            pltpu.sync_copy(zt_v.at[pl.ds(0, npz)], clu_hbm.at[pl.ds(n, npz)])

    return k(pos1, batch, zeros8)


def _k3_body(parts_ref, wc_ref, g1_ref, aux_ref):
    p = parts_ref[...]
    ssum = p[0] + p[1]
    cnt = jnp.maximum(ssum[:, 3:4], 1.0)
    inv = 1.0 / cnt
    pp = ssum[:, 0:3] * inv
    e = wc_ref[...]
    g1_ref[...] = (pp[:, 0:1] * e[0:1] + pp[:, 1:2] * e[1:2]
                   + pp[:, 2:3] * e[2:3])
    aux_ref[...] = jnp.concatenate([pp, inv], axis=1)


def _k3(parts4, wc):
    return pl.pallas_call(
        _k3_body,
        grid=(16,),
        in_specs=[pl.BlockSpec((2, _BM, 8), lambda i: (0, i, 0)),
                  pl.BlockSpec((8, 128), lambda i: (0, 0))],
        out_specs=[pl.BlockSpec((_BM, 128), lambda i: (i, 0)),
                   pl.BlockSpec((_BM, 4), lambda i: (i, 0))],
        out_shape=[jax.ShapeDtypeStruct((_MP, 128), jnp.float32),
                   jax.ShapeDtypeStruct((_MP, 4), jnp.float32)],
    )(parts4, wc)


def _sc_gather(tab, clu):
    npad = clu.shape[0]
    rows = npad // 16
    nch = 28
    ch = rows // nch
    mrows = _MP // 16

    @functools.partial(
        pl.kernel,
        out_type=jax.ShapeDtypeStruct((npad, 128), jnp.float32),
        mesh=_get_mesh(),
        compiler_params=pltpu.CompilerParams(use_tc_tiling_on_sc=False),
        scratch_types=[pltpu.VMEM((rows,), jnp.int32),
                       [pltpu.VMEM((ch, 64), jnp.float32)] * 2,
                       [pltpu.SemaphoreType.DMA] * 2,
                       [pltpu.SemaphoreType.DMA] * 2,
                       pltpu.VMEM_SHARED((_MP, 64), jnp.float32)],
    )
    def k(tab_hbm, clu_hbm, out_hbm, idx_v, bufs, gsems, osems, tab_s):
        c = lax.axis_index("c")
        s = lax.axis_index("s")
        base = s * rows
        pltpu.sync_copy(clu_hbm.at[pl.ds(base, rows)], idx_v)
        mb = s * mrows

        @pl.when(c == 0)
        def _():
            pltpu.sync_copy(tab_hbm.at[pl.ds(mb, mrows), pl.ds(0, 64)],
                            tab_s.at[pl.ds(mb, mrows)])

        @pl.when(c == 1)
        def _():
            pltpu.sync_copy(tab_hbm.at[pl.ds(mb, mrows), pl.ds(64, 64)],
                            tab_s.at[pl.ds(mb, mrows)])

        plsc.subcore_barrier()

        def start_gather(j):
            return pltpu.async_copy(
                tab_s.at[idx_v.at[pl.ds(j * ch, ch)]],
                bufs[j % 2], gsems[j % 2])

        def start_out(j):
            b = bufs[j % 2]
            sem = osems[j % 2]
            off = base + j * ch

            @pl.when(c == 0)
            def _():
                pltpu.async_copy(b, out_hbm.at[pl.ds(off, ch), pl.ds(0, 64)],
                                 sem)

            @pl.when(c == 1)
            def _():
                pltpu.async_copy(b, out_hbm.at[pl.ds(off, ch), pl.ds(64, 64)],
                                 sem)

        def drain_out(j):
            pltpu.make_async_copy(
                out_hbm.at[pl.ds(base, ch), pl.ds(0, 64)],
                bufs[j % 2], osems[j % 2]).wait()

        gd = [None, None]
        started = [False, False]
        gd[0] = start_gather(0)
        for j in range(nch):
            nxt = (j + 1) % 2
            if j + 1 < nch:
                if started[nxt]:
                    drain_out(j - 1)
                gd[nxt] = start_gather(j + 1)
            gd[j % 2].wait()
            start_out(j)
            started[j % 2] = True
        drain_out(nch - 2)
        drain_out(nch - 1)

    return k(tab, clu)


def _k5_body(nreal, x_ref, p_ref, g_ref, wa_ref, w2_ref, e_ref, out_ref):
    i = pl.program_id(0)
    e = e_ref[...]
    pt = p_ref[...]
    t = jnp.dot(x_ref[...], wa_ref[...], preferred_element_type=jnp.float32)
    t = t + g_ref[...] + e[3:4]
    t = t + lax.dot_general(pt, e_ref[...],
                            (((0,), (0,)), ((), ())),
                            preferred_element_type=jnp.float32)
    a = jnp.maximum(t, 0.0)
    h = jnp.dot(a, w2_ref[...], preferred_element_type=jnp.float32) + e[4:5]
    h = jnp.maximum(h, 0.0)
    row = i * _R + lax.broadcasted_iota(jnp.int32, (_R, 1), 0)
    out_ref[...] = jnp.where(row < nreal, h, 0.0)


def _k5(x, pos, g1c, wa, w2, e, npad):
    grid = npad // _R
    return pl.pallas_call(
        functools.partial(_k5_body, x.shape[0]),
        grid=(grid,),
        in_specs=[pl.BlockSpec((_R, 128), lambda i: (i, 0)),
                  pl.BlockSpec((8, _R), lambda i: (0, i)),
                  pl.BlockSpec((_R, 128), lambda i: (i, 0)),
                  pl.BlockSpec((128, 128), lambda i: (0, 0)),
                  pl.BlockSpec((128, 128), lambda i: (0, 0)),
                  pl.BlockSpec((8, 128), lambda i: (0, 0))],
        out_specs=pl.BlockSpec((_R, 128), lambda i: (i, 0)),
        out_shape=jax.ShapeDtypeStruct((npad, 128), jnp.float32),
    )(x, pos, g1c, wa, w2, e)


def _k6(h, clu, zeros64):
    npad = clu.shape[0]
    rows = npad // 16
    nch = 28
    ch = rows // nch

    @functools.partial(
        pl.kernel,
        out_type=jax.ShapeDtypeStruct((_MP, 128), jnp.float32),
        mesh=_get_mesh(),
        compiler_params=pltpu.CompilerParams(use_tc_tiling_on_sc=False),
        scratch_types=[pltpu.VMEM((rows,), jnp.int32),
                       [pltpu.VMEM((ch, 64), jnp.float32)] * 2,
                       [pltpu.SemaphoreType.DMA] * 2,
                       pltpu.VMEM_SHARED((_MP, 64), jnp.float32)],
    )
    def k(h_hbm, clu_hbm, z_hbm, out_hbm, idx_v, bufs, hsems, acc_s):
        c = lax.axis_index("c")
        s = lax.axis_index("s")

        @pl.when(s == 0)
        def _():
            pltpu.sync_copy(z_hbm, acc_s)

        base = s * rows
        pltpu.sync_copy(clu_hbm.at[pl.ds(base, rows)], idx_v)

        def start_load(j):
            off = base + j * ch
            b = bufs[j % 2]
            sem = hsems[j % 2]

            @pl.when(c == 0)
            def _():
                pltpu.async_copy(h_hbm.at[pl.ds(off, ch), pl.ds(0, 64)],
                                 b, sem)

            @pl.when(c == 1)
            def _():
                pltpu.async_copy(h_hbm.at[pl.ds(off, ch), pl.ds(64, 64)],
                                 b, sem)

        def drain(j):
            pltpu.make_async_copy(
                h_hbm.at[pl.ds(base, ch), pl.ds(0, 64)],
                bufs[j % 2], hsems[j % 2]).wait()

        start_load(0)
        plsc.subcore_barrier()
        for j in range(nch):
            if j + 1 < nch:
                start_load(j + 1)
            drain(j)
            pltpu.sync_copy(bufs[j % 2],
                            acc_s.at[idx_v.at[pl.ds(j * ch, ch)]], add=True)
        plsc.subcore_barrier()

        @pl.when(s == 0)
        def _():
            @pl.when(c == 0)
            def _():
                pltpu.sync_copy(acc_s, out_hbm.at[:, pl.ds(0, 64)])

            @pl.when(c == 1)
            def _():
                pltpu.sync_copy(acc_s, out_hbm.at[:, pl.ds(64, 64)])

    return k(h, clu, zeros64)


def _k7_body(sh_ref, aux_ref, wu_ref, wpu_ref, e_ref, z_ref):
    a = aux_ref[...]
    inv = a[:, 3:4]
    u = jnp.dot(sh_ref[...] * inv, wu_ref[...],
                preferred_element_type=jnp.float32)
    e = e_ref[...]
    u = jnp.maximum(u + e[3:4], 0.0)
    z = jnp.dot(u, wpu_ref[...], preferred_element_type=jnp.float32)
    z_ref[...] = (z + a[:, 0:1] * e[0:1] + a[:, 1:2] * e[1:2]
                  + a[:, 2:3] * e[2:3])


def _k7(segh, aux, wu, wpu, e):
    return pl.pallas_call(
        _k7_body,
        grid=(16,),
        in_specs=[pl.BlockSpec((_BM, 128), lambda i: (i, 0)),
                  pl.BlockSpec((_BM, 4), lambda i: (i, 0)),
                  pl.BlockSpec((128, 128), lambda i: (0, 0)),
                  pl.BlockSpec((128, 128), lambda i: (0, 0)),
                  pl.BlockSpec((8, 128), lambda i: (0, 0))],
        out_specs=pl.BlockSpec((_BM, 128), lambda i: (i, 0)),
        out_shape=jax.ShapeDtypeStruct((_MP, 128), jnp.float32),
    )(segh, aux, wu, wpu, e)


def _k9_body(x_ref, p_ref, zc_ref, wa_ref, w2_ref, e_ref, out_ref):
    e = e_ref[...]
    pt = p_ref[...]
    t = jnp.dot(x_ref[...], wa_ref[...], preferred_element_type=jnp.float32)
    t = t + zc_ref[...] + e[3:4]
    t = t + lax.dot_general(pt, e_ref[...],
                            (((0,), (0,)), ((), ())),
                            preferred_element_type=jnp.float32)
    a = jnp.maximum(t, 0.0)
    o = jnp.dot(a, w2_ref[...], preferred_element_type=jnp.float32) + e[4:5]
    out_ref[...] = jnp.maximum(o, 0.0)


def _k9(x, pos, zc, wa, w2, e, npad):
    grid = npad // _R
    n = x.shape[0]
    return pl.pallas_call(
        _k9_body,
        grid=(grid,),
        in_specs=[pl.BlockSpec((_R, 128), lambda i: (i, 0)),
                  pl.BlockSpec((8, _R), lambda i: (0, i)),
                  pl.BlockSpec((_R, 128), lambda i: (i, 0)),
                  pl.BlockSpec((128, 128), lambda i: (0, 0)),
                  pl.BlockSpec((128, 128), lambda i: (0, 0)),
                  pl.BlockSpec((8, 128), lambda i: (0, 0))],
        out_specs=pl.BlockSpec((_R, 128), lambda i: (i, 0)),
        out_shape=jax.ShapeDtypeStruct((n, 128), jnp.float32),
    )(x, pos, zc, wa, w2, e)


def kernel(x, pos, batch, W_pre1, b_pre1, W_pre2, b_pre2,
           W_unet, b_unet, W_post1, b_post1, W_post2, b_post2):
    N = x.shape[0]
    npad = -(-N // _R) * _R

    pad3 = jnp.zeros((3, 128), jnp.float32)
    e_pre1 = jnp.concatenate([W_pre1[131:134], jnp.zeros((5, 128))], axis=0)
    e5 = jnp.concatenate([W_pre1[128:131], b_pre1[None], b_pre2[None], pad3],
                         axis=0)
    e7 = jnp.concatenate([W_post1[131:134], b_unet[None],
                          jnp.zeros((4, 128))], axis=0)
    e9 = jnp.concatenate([W_post1[128:131], b_post1[None], b_post2[None],
                          pad3], axis=0)

    pos_t8 = jnp.zeros((8, N), jnp.float32).at[:3].set(pos.T)
    clu, parts4 = _k2(pos_t8.reshape(-1), batch,
                      jnp.zeros((_MP, 8), jnp.float32), npad)
    g1, aux = _k3(parts4, e_pre1)
    g1c = _sc_gather(g1, clu)
    h = _k5(x, pos_t8, g1c, W_pre1[:128], W_pre2, e5, npad)
    segh = _k6(h, clu, jnp.zeros((_MP, 64), jnp.float32))
    z = _k7(segh, aux, W_unet, W_post1[134:262], e7)
    zc = _sc_gather(z, clu)
    return _k9(x, pos_t8, zc, W_post1[:128], W_post2, e9, npad)


# speedup vs baseline: 4.2812x; 1.0856x over previous
import functools

import jax
import jax.numpy as jnp
from jax import lax
from jax.experimental import pallas as pl
from jax.experimental.pallas import tpu as pltpu
from jax.experimental.pallas import tpu_sc as plsc

_VOXEL = 0.1
_SIDE = 11
_NBATCH = 16
_M = _NBATCH * _SIDE ** 3
_MP = 21504
_BM = _MP // 16
_R = 3584
_NW = 32


@functools.cache
def _get_mesh():
    return plsc.VectorSubcoreMesh(core_axis_name="c", subcore_axis_name="s",
                                  num_cores=2, num_subcores=16)


def _k2(pos1, batch, zeros8, npad):
    n = batch.shape[0]
    bw = (n // (_NW * 16)) * 16
    tail = n - _NW * bw
    ch = bw + tail
    n1 = bw // 16
    n2 = ch // 16
    npz = npad - n

    @functools.partial(
        pl.kernel,
        out_type=[jax.ShapeDtypeStruct((npad,), jnp.int32),
                  jax.ShapeDtypeStruct((2, _MP, 8), jnp.float32)],
        mesh=_get_mesh(),
        compiler_params=pltpu.CompilerParams(use_tc_tiling_on_sc=False,
                                             needs_layout_passes=False),
        scratch_types=[pltpu.VMEM((ch,), jnp.int32),
                       pltpu.VMEM((ch, 8), jnp.float32),
                       pltpu.VMEM((ch,), jnp.int32),
                       pltpu.VMEM((ch * 3,), jnp.float32),
                       pltpu.VMEM((max(npz, 16),), jnp.int32),
                       pltpu.VMEM_SHARED((_MP, 8), jnp.float32)],
    )
    def k(pos_hbm, bat_hbm, z_hbm, clu_hbm, parts_hbm,
          idx_v, val_v, bat_v, pos_v, zt_v, acc_s):
        c = lax.axis_index("c")
        s = lax.axis_index("s")
        wid = s * 2 + c
        base = wid * bw

        @pl.when(s == 0)
        def _():
            pltpu.sync_copy(z_hbm, acc_s)

        for d in range(3):
            pltpu.sync_copy(pos_hbm.at[pl.ds(d * n + base, ch)],
                            pos_v.at[pl.ds(d * ch, ch)])
        pltpu.sync_copy(bat_hbm.at[pl.ds(base, ch)], bat_v)
        pltpu.sync_copy(z_hbm.at[pl.ds(0, ch)], val_v)

        lanes = lax.iota(jnp.int32, 16)
        is_last = (jnp.zeros((16,), jnp.int32) + wid) == (_NW - 1)
        big = jnp.float32(8388608.0)
        ones16 = jnp.full((16,), 1.0, jnp.float32)

        def step(v, masked):
            r0 = v * 16
            rows = r0 + lanes
            b16 = bat_v[pl.ds(r0, 16)]
            cs = []
            m = is_last if masked else None
            for d in range(3):
                col = jnp.full((16,), d, jnp.int32)
                xd = pos_v[pl.ds(d * ch + r0, 16)]
                plsc.store_scatter(val_v, [rows, col], xd, mask=m)
                rd = xd / jnp.float32(_VOXEL)
                cs.append(((rd + big) - big).astype(jnp.int32))
            plsc.store_scatter(val_v, [rows, jnp.full((16,), 3, jnp.int32)],
                               ones16, mask=m)
            key = ((b16 * _SIDE + cs[0]) * _SIDE + cs[1]) * _SIDE + cs[2]
            idx_v[pl.ds(r0, 16)] = key

        lax.fori_loop(0, n1, lambda v, _: (step(v, False), 0)[1], 0)
        lax.fori_loop(n1, n2, lambda v, _: (step(v, True), 0)[1], 0)

        plsc.subcore_barrier()
        pltpu.sync_copy(val_v, acc_s.at[idx_v], add=True)
        plsc.subcore_barrier()

        @pl.when(s == 0)
        def _():
            pltpu.sync_copy(acc_s, parts_hbm.at[c])

        pltpu.sync_copy(idx_v.at[pl.ds(0, bw)], clu_hbm.at[pl.ds(base, bw)])

        @pl.when(wid == _NW - 1)
        def _():
            pltpu.sync_copy(idx_v.at[pl.ds(bw, tail)],
                            clu_hbm.at[pl.ds(_NW * bw, tail)])
            for t in range(npz // 16):
                zt_v[pl.ds(16 * t, 16)] = jnp.zeros((16,), jnp.int32)
            pltpu.sync_copy(zt_v.at[pl.ds(0, npz)], clu_hbm.at[pl.ds(n, npz)])

    return k(pos1, batch, zeros8)


def _k3_body(parts_ref, wc_ref, g1_ref, aux_ref):
    p = parts_ref[...]
    ssum = p[0] + p[1]
    cnt = jnp.maximum(ssum[:, 3:4], 1.0)
    inv = 1.0 / cnt
    pp = ssum[:, 0:3] * inv
    e = wc_ref[...]
    g1_ref[...] = (pp[:, 0:1] * e[0:1] + pp[:, 1:2] * e[1:2]
                   + pp[:, 2:3] * e[2:3])
    aux_ref[...] = jnp.concatenate([pp, inv], axis=1)


def _k3(parts4, wc):
    return pl.pallas_call(
        _k3_body,
        grid=(16,),
        in_specs=[pl.BlockSpec((2, _BM, 8), lambda i: (0, i, 0)),
                  pl.BlockSpec((8, 128), lambda i: (0, 0))],
        out_specs=[pl.BlockSpec((_BM, 128), lambda i: (i, 0)),
                   pl.BlockSpec((_BM, 4), lambda i: (i, 0))],
        out_shape=[jax.ShapeDtypeStruct((_MP, 128), jnp.float32),
                   jax.ShapeDtypeStruct((_MP, 4), jnp.float32)],
    )(parts4, wc)


def _sc_gather(tab, clu):
    npad = clu.shape[0]
    rows = npad // 16
    nch = 28
    ch = rows // nch
    mrows = _MP // 16

    @functools.partial(
        pl.kernel,
        out_type=jax.ShapeDtypeStruct((npad, 128), jnp.float32),
        mesh=_get_mesh(),
        compiler_params=pltpu.CompilerParams(use_tc_tiling_on_sc=False),
        scratch_types=[pltpu.VMEM((rows,), jnp.int32),
                       [pltpu.VMEM((ch, 64), jnp.float32)] * 2,
                       [pltpu.SemaphoreType.DMA] * 2,
                       [pltpu.SemaphoreType.DMA] * 2,
                       pltpu.VMEM_SHARED((_MP, 64), jnp.float32)],
    )
    def k(tab_hbm, clu_hbm, out_hbm, idx_v, bufs, gsems, osems, tab_s):
        c = lax.axis_index("c")
        s = lax.axis_index("s")
        base = s * rows
        pltpu.sync_copy(clu_hbm.at[pl.ds(base, rows)], idx_v)
        mb = s * mrows

        @pl.when(c == 0)
        def _():
            pltpu.sync_copy(tab_hbm.at[pl.ds(mb, mrows), pl.ds(0, 64)],
                            tab_s.at[pl.ds(mb, mrows)])

        @pl.when(c == 1)
        def _():
            pltpu.sync_copy(tab_hbm.at[pl.ds(mb, mrows), pl.ds(64, 64)],
                            tab_s.at[pl.ds(mb, mrows)])

        plsc.subcore_barrier()

        def start_gather(j):
            return pltpu.async_copy(
                tab_s.at[idx_v.at[pl.ds(j * ch, ch)]],
                bufs[j % 2], gsems[j % 2])

        def start_out(j):
            b = bufs[j % 2]
            sem = osems[j % 2]
            off = base + j * ch

            @pl.when(c == 0)
            def _():
                pltpu.async_copy(b, out_hbm.at[pl.ds(off, ch), pl.ds(0, 64)],
                                 sem)

            @pl.when(c == 1)
            def _():
                pltpu.async_copy(b, out_hbm.at[pl.ds(off, ch), pl.ds(64, 64)],
                                 sem)

        def drain_out(j):
            pltpu.make_async_copy(
                out_hbm.at[pl.ds(base, ch), pl.ds(0, 64)],
                bufs[j % 2], osems[j % 2]).wait()

        gd = [None, None]
        started = [False, False]
        gd[0] = start_gather(0)
        for j in range(nch):
            nxt = (j + 1) % 2
            if j + 1 < nch:
                if started[nxt]:
                    drain_out(j - 1)
                gd[nxt] = start_gather(j + 1)
            gd[j % 2].wait()
            start_out(j)
            started[j % 2] = True
        drain_out(nch - 2)
        drain_out(nch - 1)

    return k(tab, clu)


def _k5_body(nreal, x_ref, p_ref, g_ref, wa_ref, w2_ref, e_ref, out_ref):
    i = pl.program_id(0)
    e = e_ref[...]
    pt = p_ref[...]
    t = jnp.dot(x_ref[...], wa_ref[...], preferred_element_type=jnp.float32)
    t = t + g_ref[...] + e[3:4]
    t = t + lax.dot_general(pt, e_ref[...],
                            (((0,), (0,)), ((), ())),
                            preferred_element_type=jnp.float32)
    a = jnp.maximum(t, 0.0)
    h = jnp.dot(a, w2_ref[...], preferred_element_type=jnp.float32) + e[4:5]
    h = jnp.maximum(h, 0.0)
    row = i * _R + lax.broadcasted_iota(jnp.int32, (_R, 1), 0)
    out_ref[...] = jnp.where(row < nreal, h, 0.0)


def _k5(x, pos, g1c, wa, w2, e, npad):
    grid = npad // _R
    return pl.pallas_call(
        functools.partial(_k5_body, x.shape[0]),
        grid=(grid,),
        in_specs=[pl.BlockSpec((_R, 128), lambda i: (i, 0)),
                  pl.BlockSpec((8, _R), lambda i: (0, i)),
                  pl.BlockSpec((_R, 128), lambda i: (i, 0)),
                  pl.BlockSpec((128, 128), lambda i: (0, 0)),
                  pl.BlockSpec((128, 128), lambda i: (0, 0)),
                  pl.BlockSpec((8, 128), lambda i: (0, 0))],
        out_specs=pl.BlockSpec((_R, 128), lambda i: (i, 0)),
        out_shape=jax.ShapeDtypeStruct((npad, 128), jnp.float32),
    )(x, pos, g1c, wa, w2, e)


def _k6(h, clu, zeros64):
    npad = clu.shape[0]
    rows = npad // 16
    nch = 28
    ch = rows // nch

    @functools.partial(
        pl.kernel,
        out_type=jax.ShapeDtypeStruct((_MP, 128), jnp.float32),
        mesh=_get_mesh(),
        compiler_params=pltpu.CompilerParams(use_tc_tiling_on_sc=False),
        scratch_types=[pltpu.VMEM((rows,), jnp.int32),
                       [pltpu.VMEM((ch, 64), jnp.float32)] * 2,
                       [pltpu.SemaphoreType.DMA] * 2,
                       pltpu.VMEM_SHARED((_MP, 64), jnp.float32)],
    )
    def k(h_hbm, clu_hbm, z_hbm, out_hbm, idx_v, bufs, hsems, acc_s):
        c = lax.axis_index("c")
        s = lax.axis_index("s")

        @pl.when(s == 0)
        def _():
            pltpu.sync_copy(z_hbm, acc_s)

        base = s * rows
        pltpu.sync_copy(clu_hbm.at[pl.ds(base, rows)], idx_v)

        def start_load(j):
            off = base + j * ch
            b = bufs[j % 2]
            sem = hsems[j % 2]

            @pl.when(c == 0)
            def _():
                pltpu.async_copy(h_hbm.at[pl.ds(off, ch), pl.ds(0, 64)],
                                 b, sem)

            @pl.when(c == 1)
            def _():
                pltpu.async_copy(h_hbm.at[pl.ds(off, ch), pl.ds(64, 64)],
                                 b, sem)

        def drain(j):
            pltpu.make_async_copy(
                h_hbm.at[pl.ds(base, ch), pl.ds(0, 64)],
                bufs[j % 2], hsems[j % 2]).wait()

        start_load(0)
        plsc.subcore_barrier()
        for j in range(nch):
            if j + 1 < nch:
                start_load(j + 1)
            drain(j)
            pltpu.sync_copy(bufs[j % 2],
                            acc_s.at[idx_v.at[pl.ds(j * ch, ch)]], add=True)
        plsc.subcore_barrier()

        @pl.when(s == 0)
        def _():
            @pl.when(c == 0)
            def _():
                pltpu.sync_copy(acc_s, out_hbm.at[:, pl.ds(0, 64)])

            @pl.when(c == 1)
            def _():
                pltpu.sync_copy(acc_s, out_hbm.at[:, pl.ds(64, 64)])

    return k(h, clu, zeros64)


def _k7_body(sh_ref, aux_ref, wu_ref, wpu_ref, e_ref, z_ref):
    a = aux_ref[...]
    inv = a[:, 3:4]
    u = jnp.dot(sh_ref[...] * inv, wu_ref[...],
                preferred_element_type=jnp.float32)
    e = e_ref[...]
    u = jnp.maximum(u + e[3:4], 0.0)
    z = jnp.dot(u, wpu_ref[...], preferred_element_type=jnp.float32)
    z_ref[...] = (z + a[:, 0:1] * e[0:1] + a[:, 1:2] * e[1:2]
                  + a[:, 2:3] * e[2:3])


def _k7(segh, aux, wu, wpu, e):
    return pl.pallas_call(
        _k7_body,
        grid=(16,),
        in_specs=[pl.BlockSpec((_BM, 128), lambda i: (i, 0)),
                  pl.BlockSpec((_BM, 4), lambda i: (i, 0)),
                  pl.BlockSpec((128, 128), lambda i: (0, 0)),
                  pl.BlockSpec((128, 128), lambda i: (0, 0)),
                  pl.BlockSpec((8, 128), lambda i: (0, 0))],
        out_specs=pl.BlockSpec((_BM, 128), lambda i: (i, 0)),
        out_shape=jax.ShapeDtypeStruct((_MP, 128), jnp.float32),
    )(segh, aux, wu, wpu, e)


def _k9_body(x_ref, p_ref, zc_ref, wa_ref, w2_ref, e_ref, out_ref):
    e = e_ref[...]
    pt = p_ref[...]
    t = jnp.dot(x_ref[...], wa_ref[...], preferred_element_type=jnp.float32)
    t = t + zc_ref[...] + e[3:4]
    t = t + lax.dot_general(pt, e_ref[...],
                            (((0,), (0,)), ((), ())),
                            preferred_element_type=jnp.float32)
    a = jnp.maximum(t, 0.0)
    o = jnp.dot(a, w2_ref[...], preferred_element_type=jnp.float32) + e[4:5]
    out_ref[...] = jnp.maximum(o, 0.0)


def _k9(x, pos, zc, wa, w2, e, npad):
    grid = npad // _R
    n = x.shape[0]
    return pl.pallas_call(
        _k9_body,
        grid=(grid,),
        in_specs=[pl.BlockSpec((_R, 128), lambda i: (i, 0)),
                  pl.BlockSpec((8, _R), lambda i: (0, i)),
                  pl.BlockSpec((_R, 128), lambda i: (i, 0)),
                  pl.BlockSpec((128, 128), lambda i: (0, 0)),
                  pl.BlockSpec((128, 128), lambda i: (0, 0)),
                  pl.BlockSpec((8, 128), lambda i: (0, 0))],
        out_specs=pl.BlockSpec((_R, 128), lambda i: (i, 0)),
        out_shape=jax.ShapeDtypeStruct((n, 128), jnp.float32),
    )(x, pos, zc, wa, w2, e)


def kernel(x, pos, batch, W_pre1, b_pre1, W_pre2, b_pre2,
           W_unet, b_unet, W_post1, b_post1, W_post2, b_post2):
    N = x.shape[0]
    npad = -(-N // _R) * _R

    pad3 = jnp.zeros((3, 128), jnp.float32)
    e_pre1 = jnp.concatenate([W_pre1[131:134], jnp.zeros((5, 128))], axis=0)
    e5 = jnp.concatenate([W_pre1[128:131], b_pre1[None], b_pre2[None], pad3],
                         axis=0)
    e7 = jnp.concatenate([W_post1[131:134], b_unet[None],
                          jnp.zeros((4, 128))], axis=0)
    e9 = jnp.concatenate([W_post1[128:131], b_post1[None], b_post2[None],
                          pad3], axis=0)

    pos_t8 = jnp.zeros((8, N), jnp.float32).at[:3].set(pos.T)
    clu, parts4 = _k2(pos_t8.reshape(-1), batch,
                      jnp.zeros((_MP, 8), jnp.float32), npad)
    g1, aux = _k3(parts4, e_pre1)
    g1c = _sc_gather(g1, clu)
    h = _k5(x, pos_t8, g1c, W_pre1[:128], W_pre2, e5, npad)
    segh = _k6(h, clu, jnp.zeros((_MP, 64), jnp.float32))
    z = _k7(segh, aux, W_unet, W_post1[134:262], e7)
    zc = _sc_gather(z, clu)
    return _k9(x, pos_t8, zc, W_post1[:128], W_post2, e9, npad)


# speedup vs baseline: 4.4320x; 1.0352x over previous
import functools

import jax
import jax.numpy as jnp
from jax import lax
from jax.experimental import pallas as pl
from jax.experimental.pallas import tpu as pltpu
from jax.experimental.pallas import tpu_sc as plsc

_VOXEL = 0.1
_SIDE = 11
_NBATCH = 16
_M = _NBATCH * _SIDE ** 3
_MP = 21504
_BM = _MP // 16
_R = 7168
_NW = 32


@functools.cache
def _get_mesh():
    return plsc.VectorSubcoreMesh(core_axis_name="c", subcore_axis_name="s",
                                  num_cores=2, num_subcores=16)


def _k2(pos1, batch, zeros8, npad):
    n = batch.shape[0]
    bw = (n // (_NW * 16)) * 16
    tail = n - _NW * bw
    ch = bw + tail
    n1 = bw // 16
    n2 = ch // 16
    npz = npad - n

    @functools.partial(
        pl.kernel,
        out_type=[jax.ShapeDtypeStruct((npad,), jnp.int32),
                  jax.ShapeDtypeStruct((2, _MP, 8), jnp.float32)],
        mesh=_get_mesh(),
        compiler_params=pltpu.CompilerParams(use_tc_tiling_on_sc=False,
                                             needs_layout_passes=False),
        scratch_types=[pltpu.VMEM((ch,), jnp.int32),
                       pltpu.VMEM((ch, 8), jnp.float32),
                       pltpu.VMEM((ch,), jnp.int32),
                       pltpu.VMEM((ch * 3,), jnp.float32),
                       pltpu.VMEM((max(npz, 16),), jnp.int32),
                       pltpu.VMEM_SHARED((_MP, 8), jnp.float32)],
    )
    def k(pos_hbm, bat_hbm, z_hbm, clu_hbm, parts_hbm,
          idx_v, val_v, bat_v, pos_v, zt_v, acc_s):
        c = lax.axis_index("c")
        s = lax.axis_index("s")
        wid = s * 2 + c
        base = wid * bw

        @pl.when(s == 0)
        def _():
            pltpu.sync_copy(z_hbm, acc_s)

        for d in range(3):
            pltpu.sync_copy(pos_hbm.at[pl.ds(d * n + base, ch)],
                            pos_v.at[pl.ds(d * ch, ch)])
        pltpu.sync_copy(bat_hbm.at[pl.ds(base, ch)], bat_v)
        pltpu.sync_copy(z_hbm.at[pl.ds(0, ch)], val_v)

        lanes = lax.iota(jnp.int32, 16)
        is_last = (jnp.zeros((16,), jnp.int32) + wid) == (_NW - 1)
        big = jnp.float32(8388608.0)
        ones16 = jnp.full((16,), 1.0, jnp.float32)

        def step(v, masked):
            r0 = v * 16
            rows = r0 + lanes
            b16 = bat_v[pl.ds(r0, 16)]
            cs = []
            m = is_last if masked else None
            for d in range(3):
                col = jnp.full((16,), d, jnp.int32)
                xd = pos_v[pl.ds(d * ch + r0, 16)]
                plsc.store_scatter(val_v, [rows, col], xd, mask=m)
                rd = xd / jnp.float32(_VOXEL)
                cs.append(((rd + big) - big).astype(jnp.int32))
            plsc.store_scatter(val_v, [rows, jnp.full((16,), 3, jnp.int32)],
                               ones16, mask=m)
            key = ((b16 * _SIDE + cs[0]) * _SIDE + cs[1]) * _SIDE + cs[2]
            idx_v[pl.ds(r0, 16)] = key

        lax.fori_loop(0, n1, lambda v, _: (step(v, False), 0)[1], 0)
        lax.fori_loop(n1, n2, lambda v, _: (step(v, True), 0)[1], 0)

        plsc.subcore_barrier()
        pltpu.sync_copy(val_v, acc_s.at[idx_v], add=True)
        plsc.subcore_barrier()

        @pl.when(s == 0)
        def _():
            pltpu.sync_copy(acc_s, parts_hbm.at[c])

        pltpu.sync_copy(idx_v.at[pl.ds(0, bw)], clu_hbm.at[pl.ds(base, bw)])

        @pl.when(wid == _NW - 1)
        def _():
            pltpu.sync_copy(idx_v.at[pl.ds(bw, tail)],
                            clu_hbm.at[pl.ds(_NW * bw, tail)])
            for t in range(npz // 16):
                zt_v[pl.ds(16 * t, 16)] = jnp.zeros((16,), jnp.int32)
            pltpu.sync_copy(zt_v.at[pl.ds(0, npz)], clu_hbm.at[pl.ds(n, npz)])

    return k(pos1, batch, zeros8)


def _k3_body(parts_ref, wc_ref, g1_ref, aux_ref):
    p = parts_ref[...]
    ssum = p[0] + p[1]
    cnt = jnp.maximum(ssum[:, 3:4], 1.0)
    inv = 1.0 / cnt
    pp = ssum[:, 0:3] * inv
    e = wc_ref[...]
    g1_ref[...] = (pp[:, 0:1] * e[0:1] + pp[:, 1:2] * e[1:2]
                   + pp[:, 2:3] * e[2:3])
    aux_ref[...] = jnp.concatenate([pp, inv], axis=1)


def _k3(parts4, wc):
    return pl.pallas_call(
        _k3_body,
        grid=(16,),
        in_specs=[pl.BlockSpec((2, _BM, 8), lambda i: (0, i, 0)),
                  pl.BlockSpec((8, 128), lambda i: (0, 0))],
        out_specs=[pl.BlockSpec((_BM, 128), lambda i: (i, 0)),
                   pl.BlockSpec((_BM, 4), lambda i: (i, 0))],
        out_shape=[jax.ShapeDtypeStruct((_MP, 128), jnp.float32),
                   jax.ShapeDtypeStruct((_MP, 4), jnp.float32)],
    )(parts4, wc)


def _sc_gather(tab, clu):
    npad = clu.shape[0]
    rows = npad // 16
    nch = 28
    ch = rows // nch
    mrows = _MP // 16

    @functools.partial(
        pl.kernel,
        out_type=jax.ShapeDtypeStruct((npad, 128), jnp.float32),
        mesh=_get_mesh(),
        compiler_params=pltpu.CompilerParams(use_tc_tiling_on_sc=False),
        scratch_types=[pltpu.VMEM((rows,), jnp.int32),
                       [pltpu.VMEM((ch, 64), jnp.float32)] * 2,
                       [pltpu.SemaphoreType.DMA] * 2,
                       [pltpu.SemaphoreType.DMA] * 2,
                       pltpu.VMEM_SHARED((_MP, 64), jnp.float32)],
    )
    def k(tab_hbm, clu_hbm, out_hbm, idx_v, bufs, gsems, osems, tab_s):
        c = lax.axis_index("c")
        s = lax.axis_index("s")
        base = s * rows
        pltpu.sync_copy(clu_hbm.at[pl.ds(base, rows)], idx_v)
        mb = s * mrows

        @pl.when(c == 0)
        def _():
            pltpu.sync_copy(tab_hbm.at[pl.ds(mb, mrows), pl.ds(0, 64)],
                            tab_s.at[pl.ds(mb, mrows)])

        @pl.when(c == 1)
        def _():
            pltpu.sync_copy(tab_hbm.at[pl.ds(mb, mrows), pl.ds(64, 64)],
                            tab_s.at[pl.ds(mb, mrows)])

        plsc.subcore_barrier()

        def start_gather(j):
            return pltpu.async_copy(
                tab_s.at[idx_v.at[pl.ds(j * ch, ch)]],
                bufs[j % 2], gsems[j % 2])

        def start_out(j):
            b = bufs[j % 2]
            sem = osems[j % 2]
            off = base + j * ch

            @pl.when(c == 0)
            def _():
                pltpu.async_copy(b, out_hbm.at[pl.ds(off, ch), pl.ds(0, 64)],
                                 sem)

            @pl.when(c == 1)
            def _():
                pltpu.async_copy(b, out_hbm.at[pl.ds(off, ch), pl.ds(64, 64)],
                                 sem)

        def drain_out(j):
            pltpu.make_async_copy(
                out_hbm.at[pl.ds(base, ch), pl.ds(0, 64)],
                bufs[j % 2], osems[j % 2]).wait()

        gd = [None, None]
        started = [False, False]
        gd[0] = start_gather(0)
        for j in range(nch):
            nxt = (j + 1) % 2
            if j + 1 < nch:
                if started[nxt]:
                    drain_out(j - 1)
                gd[nxt] = start_gather(j + 1)
            gd[j % 2].wait()
            start_out(j)
            started[j % 2] = True
        drain_out(nch - 2)
        drain_out(nch - 1)

    return k(tab, clu)


def _k5_body(nreal, x_ref, p_ref, g_ref, wa_ref, w2_ref, e_ref, out_ref):
    i = pl.program_id(0)
    e = e_ref[...]
    pt = p_ref[...]
    t = jnp.dot(x_ref[...], wa_ref[...], preferred_element_type=jnp.float32)
    t = t + g_ref[...] + e[3:4]
    t = t + lax.dot_general(pt, e_ref[...],
                            (((0,), (0,)), ((), ())),
                            preferred_element_type=jnp.float32)
    a = jnp.maximum(t, 0.0)
    h = jnp.dot(a, w2_ref[...], preferred_element_type=jnp.float32) + e[4:5]
    h = jnp.maximum(h, 0.0)
    row = i * _R + lax.broadcasted_iota(jnp.int32, (_R, 1), 0)
    out_ref[...] = jnp.where(row < nreal, h, 0.0)


def _k5(x, pos, g1c, wa, w2, e, npad):
    grid = npad // _R
    return pl.pallas_call(
        functools.partial(_k5_body, x.shape[0]),
        grid=(grid,),
        in_specs=[pl.BlockSpec((_R, 128), lambda i: (i, 0)),
                  pl.BlockSpec((8, _R), lambda i: (0, i)),
                  pl.BlockSpec((_R, 128), lambda i: (i, 0)),
                  pl.BlockSpec((128, 128), lambda i: (0, 0)),
                  pl.BlockSpec((128, 128), lambda i: (0, 0)),
                  pl.BlockSpec((8, 128), lambda i: (0, 0))],
        out_specs=pl.BlockSpec((_R, 128), lambda i: (i, 0)),
        out_shape=jax.ShapeDtypeStruct((npad, 128), jnp.float32),
    )(x, pos, g1c, wa, w2, e)


def _k6(h, clu, zeros64):
    npad = clu.shape[0]
    rows = npad // 16
    nch = 28
    ch = rows // nch

    @functools.partial(
        pl.kernel,
        out_type=jax.ShapeDtypeStruct((_MP, 128), jnp.float32),
        mesh=_get_mesh(),
        compiler_params=pltpu.CompilerParams(use_tc_tiling_on_sc=False),
        scratch_types=[pltpu.VMEM((rows,), jnp.int32),
                       [pltpu.VMEM((ch, 64), jnp.float32)] * 2,
                       [pltpu.SemaphoreType.DMA] * 2,
                       pltpu.VMEM_SHARED((_MP, 64), jnp.float32)],
    )
    def k(h_hbm, clu_hbm, z_hbm, out_hbm, idx_v, bufs, hsems, acc_s):
        c = lax.axis_index("c")
        s = lax.axis_index("s")

        @pl.when(s == 0)
        def _():
            pltpu.sync_copy(z_hbm, acc_s)

        base = s * rows
        pltpu.sync_copy(clu_hbm.at[pl.ds(base, rows)], idx_v)

        def start_load(j):
            off = base + j * ch
            b = bufs[j % 2]
            sem = hsems[j % 2]

            @pl.when(c == 0)
            def _():
                pltpu.async_copy(h_hbm.at[pl.ds(off, ch), pl.ds(0, 64)],
                                 b, sem)

            @pl.when(c == 1)
            def _():
                pltpu.async_copy(h_hbm.at[pl.ds(off, ch), pl.ds(64, 64)],
                                 b, sem)

        def drain(j):
            pltpu.make_async_copy(
                h_hbm.at[pl.ds(base, ch), pl.ds(0, 64)],
                bufs[j % 2], hsems[j % 2]).wait()

        start_load(0)
        plsc.subcore_barrier()
        for j in range(nch):
            if j + 1 < nch:
                start_load(j + 1)
            drain(j)
            pltpu.sync_copy(bufs[j % 2],
                            acc_s.at[idx_v.at[pl.ds(j * ch, ch)]], add=True)
        plsc.subcore_barrier()

        @pl.when(s == 0)
        def _():
            @pl.when(c == 0)
            def _():
                pltpu.sync_copy(acc_s, out_hbm.at[:, pl.ds(0, 64)])

            @pl.when(c == 1)
            def _():
                pltpu.sync_copy(acc_s, out_hbm.at[:, pl.ds(64, 64)])

    return k(h, clu, zeros64)


def _k7_body(sh_ref, aux_ref, wu_ref, wpu_ref, e_ref, z_ref):
    a = aux_ref[...]
    inv = a[:, 3:4]
    u = jnp.dot(sh_ref[...] * inv, wu_ref[...],
                preferred_element_type=jnp.float32)
    e = e_ref[...]
    u = jnp.maximum(u + e[3:4], 0.0)
    z = jnp.dot(u, wpu_ref[...], preferred_element_type=jnp.float32)
    z_ref[...] = (z + a[:, 0:1] * e[0:1] + a[:, 1:2] * e[1:2]
                  + a[:, 2:3] * e[2:3])


def _k7(segh, aux, wu, wpu, e):
    return pl.pallas_call(
        _k7_body,
        grid=(16,),
        in_specs=[pl.BlockSpec((_BM, 128), lambda i: (i, 0)),
                  pl.BlockSpec((_BM, 4), lambda i: (i, 0)),
                  pl.BlockSpec((128, 128), lambda i: (0, 0)),
                  pl.BlockSpec((128, 128), lambda i: (0, 0)),
                  pl.BlockSpec((8, 128), lambda i: (0, 0))],
        out_specs=pl.BlockSpec((_BM, 128), lambda i: (i, 0)),
        out_shape=jax.ShapeDtypeStruct((_MP, 128), jnp.float32),
    )(segh, aux, wu, wpu, e)


def _k9_body(x_ref, p_ref, zc_ref, wa_ref, w2_ref, e_ref, out_ref):
    e = e_ref[...]
    pt = p_ref[...]
    t = jnp.dot(x_ref[...], wa_ref[...], preferred_element_type=jnp.float32)
    t = t + zc_ref[...] + e[3:4]
    t = t + lax.dot_general(pt, e_ref[...],
                            (((0,), (0,)), ((), ())),
                            preferred_element_type=jnp.float32)
    a = jnp.maximum(t, 0.0)
    o = jnp.dot(a, w2_ref[...], preferred_element_type=jnp.float32) + e[4:5]
    out_ref[...] = jnp.maximum(o, 0.0)


def _k9(x, pos, zc, wa, w2, e, npad):
    grid = npad // _R
    n = x.shape[0]
    return pl.pallas_call(
        _k9_body,
        grid=(grid,),
        in_specs=[pl.BlockSpec((_R, 128), lambda i: (i, 0)),
                  pl.BlockSpec((8, _R), lambda i: (0, i)),
                  pl.BlockSpec((_R, 128), lambda i: (i, 0)),
                  pl.BlockSpec((128, 128), lambda i: (0, 0)),
                  pl.BlockSpec((128, 128), lambda i: (0, 0)),
                  pl.BlockSpec((8, 128), lambda i: (0, 0))],
        out_specs=pl.BlockSpec((_R, 128), lambda i: (i, 0)),
        out_shape=jax.ShapeDtypeStruct((n, 128), jnp.float32),
    )(x, pos, zc, wa, w2, e)


def kernel(x, pos, batch, W_pre1, b_pre1, W_pre2, b_pre2,
           W_unet, b_unet, W_post1, b_post1, W_post2, b_post2):
    N = x.shape[0]
    npad = -(-N // _R) * _R

    pad3 = jnp.zeros((3, 128), jnp.float32)
    e_pre1 = jnp.concatenate([W_pre1[131:134], jnp.zeros((5, 128))], axis=0)
    e5 = jnp.concatenate([W_pre1[128:131], b_pre1[None], b_pre2[None], pad3],
                         axis=0)
    e7 = jnp.concatenate([W_post1[131:134], b_unet[None],
                          jnp.zeros((4, 128))], axis=0)
    e9 = jnp.concatenate([W_post1[128:131], b_post1[None], b_post2[None],
                          pad3], axis=0)

    pos_t8 = jnp.zeros((8, N), jnp.float32).at[:3].set(pos.T)
    clu, parts4 = _k2(pos_t8.reshape(-1), batch,
                      jnp.zeros((_MP, 8), jnp.float32), npad)
    g1, aux = _k3(parts4, e_pre1)
    g1c = _sc_gather(g1, clu)
    h = _k5(x, pos_t8, g1c, W_pre1[:128], W_pre2, e5, npad)
    segh = _k6(h, clu, jnp.zeros((_MP, 64), jnp.float32))
    z = _k7(segh, aux, W_unet, W_post1[134:262], e7)
    zc = _sc_gather(z, clu)
    return _k9(x, pos_t8, zc, W_post1[:128], W_post2, e9, npad)
